# baseline (speedup 1.0000x reference)
import functools

import jax
import jax.numpy as jnp
from jax import lax
from jax.experimental import pallas as pl
from jax.experimental.pallas import tpu as pltpu
from jax.experimental.pallas import tpu_sc as plsc

NC = 2
NS = 16
NW = NC * NS
CHUNK = 128
EPS = 1e-5

F32 = jnp.float32


def _sc_mesh():
    return plsc.VectorSubcoreMesh(
        core_axis_name="c", subcore_axis_name="s",
        num_cores=NC, num_subcores=NS)


NPD = NS * 640


def _deg_body(cpt0, cpt1, dsti, ones_hbm, zer_hbm, out, dst_v, ones_v, dacc):
    c = lax.axis_index("c")
    s = lax.axis_index("s")
    nch = jnp.where(c == 0, cpt0, cpt1)
    pltpu.sync_copy(zer_hbm, dacc.at[pl.ds(s * 640, 640)])
    pltpu.sync_copy(ones_hbm, ones_v)

    @pl.when(c == 0)
    def _():
        off = pl.multiple_of(s * cpt0, GRP)
        pltpu.sync_copy(dsti.at[pl.ds(off, cpt0)], dst_v.at[pl.ds(0, cpt0)])

    @pl.when(c != 0)
    def _():
        off = pl.multiple_of(NS * cpt0 + s * cpt1, GRP)
        pltpu.sync_copy(dsti.at[pl.ds(off, cpt1)], dst_v.at[pl.ds(0, cpt1)])

    plsc.subcore_barrier()

    def body(ci, carry):
        pltpu.sync_copy(ones_v, dacc.at[dst_v.at[ci]], add=True)
        return carry

    lax.fori_loop(0, nch, body, 0)
    plsc.subcore_barrier()
    pltpu.sync_copy(dacc.at[pl.ds(s * 640, 640)],
                    out.at[c, 0, pl.ds(s * 640, 640)])


def _sc_degree(cpt0, cpt1, dsti, ones_in, zer_in):
    f = pl.kernel(
        functools.partial(_deg_body, cpt0, cpt1),
        out_type=jax.ShapeDtypeStruct((NC, 1, NPD), F32),
        mesh=_sc_mesh(),
        scratch_types=[
            pltpu.VMEM((cpt0, CHUNK), jnp.int32),
            pltpu.VMEM((CHUNK,), F32),
            pltpu.VMEM_SHARED((NPD,), F32),
        ],
    )
    return f(dsti, ones_in, zer_in)


GRP = 8


def _agg_body(cpt0, cpt1, rpw, hs, srci, dsti, zer_hbm, out,
              src_v, dst_v, rows0, acc, sem0):
    c = lax.axis_index("c")
    s = lax.axis_index("s")
    nch = jnp.where(c == 0, cpt0, cpt1)
    pltpu.sync_copy(zer_hbm, acc.at[pl.ds(s * rpw, rpw)])

    @pl.when(c == 0)
    def _():
        off = pl.multiple_of(s * cpt0, GRP)
        pltpu.sync_copy(srci.at[pl.ds(off, cpt0)], src_v.at[pl.ds(0, cpt0)])
        pltpu.sync_copy(dsti.at[pl.ds(off, cpt0)], dst_v.at[pl.ds(0, cpt0)])

    @pl.when(c != 0)
    def _():
        off = pl.multiple_of(NS * cpt0 + s * cpt1, GRP)
        pltpu.sync_copy(srci.at[pl.ds(off, cpt1)], src_v.at[pl.ds(0, cpt1)])
        pltpu.sync_copy(dsti.at[pl.ds(off, cpt1)], dst_v.at[pl.ds(0, cpt1)])

    plsc.subcore_barrier()

    def body(i, carry):
        pltpu.async_copy(hs.at[src_v.at[i]], rows0, sem0).wait()
        pltpu.sync_copy(rows0, acc.at[dst_v.at[i]], add=True)
        return carry

    lax.fori_loop(0, nch, body, 0)
    plsc.subcore_barrier()
    pltpu.sync_copy(acc.at[pl.ds(s * rpw, rpw)],
                    out.at[c, pl.ds(s * rpw, rpw)])


def _sc_aggregate(np_, cpt0, cpt1, hs, srci, dsti, zer_in):
    rpw = np_ // NS
    f = pl.kernel(
        functools.partial(_agg_body, cpt0, cpt1, rpw),
        out_type=jax.ShapeDtypeStruct((NC, np_, 128), F32),
        mesh=_sc_mesh(),
        scratch_types=[
            pltpu.VMEM((cpt0, CHUNK), jnp.int32),
            pltpu.VMEM((cpt0, CHUNK), jnp.int32),
            pltpu.VMEM((CHUNK, 128), F32),
            pltpu.VMEM_SHARED((np_, 128), F32),
            pltpu.SemaphoreType.DMA,
        ],
    )
    return f(hs, srci, dsti, zer_in)


def _gelu(v):
    return 0.5 * v * (1.0 + lax.erf(v * (2.0 ** -0.5)))


def _dot(a, b):
    return jnp.dot(a, b, preferred_element_type=F32,
                   precision=lax.Precision.HIGHEST)


def _scale_body(x_ref, w_ref, d0_ref, d1_ref, hs_ref, dinv_ref):
    deg = d0_ref[0, 0, :] + d1_ref[0, 0, :] + 1.0
    dv = lax.rsqrt(deg)
    dinv_ref[0, 0, :] = dv
    hs_ref[...] = dv[:, None] * _dot(x_ref[...], w_ref[...])


def _tc_scale(np_, nblk, r, xp, w0big, deg0, deg1):
    f = pl.pallas_call(
        _scale_body,
        grid=(nblk,),
        in_specs=[
            pl.BlockSpec((r, 512), lambda i: (i, 0)),
            pl.BlockSpec((512, 128), lambda i: (0, 0)),
            pl.BlockSpec((1, 1, r), lambda i: (i, 0, 0)),
            pl.BlockSpec((1, 1, r), lambda i: (i, 0, 0)),
        ],
        out_specs=[
            pl.BlockSpec((r, 128), lambda i: (i, 0)),
            pl.BlockSpec((1, 1, r), lambda i: (i, 0, 0)),
        ],
        out_shape=[
            jax.ShapeDtypeStruct((np_, 128), F32),
            jax.ShapeDtypeStruct((nblk, 1, r), F32),
        ],
        compiler_params=pltpu.CompilerParams(
            dimension_semantics=("arbitrary",)),
    )
    return f(xp, w0big, deg0, deg1)


def _stats_body(g, r, parts_ref, hs_ref, dinv_ref, bt_ref, batch_ref,
                s_ref, stats_ref):
    i = pl.program_id(0)
    dv = dinv_ref[0, 0, :]
    sv = dv[:, None] * (parts_ref[0] + parts_ref[1] + hs_ref[...]) \
        + bt_ref[0, :][None, :]
    s_ref[...] = sv
    b = batch_ref[0, 0, :]
    oh = (lax.broadcasted_iota(jnp.int32, (g, r), 0) == b[None, :]) \
        .astype(F32)
    a1 = _dot(oh, sv)
    a2 = _dot(oh, sv * sv)
    cnt = jnp.sum(oh, axis=1)
    new = jnp.stack([a1, a2, jnp.broadcast_to(cnt[:, None], (g, 128))])

    @pl.when(i == 0)
    def _():
        stats_ref[...] = new

    @pl.when(i > 0)
    def _():
        stats_ref[...] += new


def _tc_stats(g, np_, nblk, r, parts, hs, dinv, bt, batch3):
    f = pl.pallas_call(
        functools.partial(_stats_body, g, r),
        grid=(nblk,),
        in_specs=[
            pl.BlockSpec((2, r, 128), lambda i: (0, i, 0)),
            pl.BlockSpec((r, 128), lambda i: (i, 0)),
            pl.BlockSpec((1, 1, r), lambda i: (i, 0, 0)),
            pl.BlockSpec((1, 128), lambda i: (0, 0)),
            pl.BlockSpec((1, 1, r), lambda i: (i, 0, 0)),
        ],
        out_specs=[
            pl.BlockSpec((r, 128), lambda i: (i, 0)),
            pl.BlockSpec((3, g, 128), lambda i: (0, 0, 0)),
        ],
        out_shape=[
            jax.ShapeDtypeStruct((np_, 128), F32),
            jax.ShapeDtypeStruct((3, g, 128), F32),
        ],
        compiler_params=pltpu.CompilerParams(
            dimension_semantics=("arbitrary",)),
    )
    return f(parts, hs, dinv, bt, batch3)


def _norm_scale_off(stats, gw, gb, gms):
    cnt = jnp.maximum(stats[2], 1.0)
    mean = stats[0] / cnt
    e2 = stats[1] / cnt
    ms = gms[0, :][None, :]
    var = e2 - mean * mean * ms * (2.0 - ms)
    rs = lax.rsqrt(var + EPS)
    w = gw[0, :][None, :]
    scale = w * rs
    off = gb[0, :][None, :] - w * ms * mean * rs
    return scale, off


def _row_gather(batch_ref, g, r, scale, off):
    b = batch_ref[0, 0, :]
    oht = (b[:, None] == lax.broadcasted_iota(jnp.int32, (r, g), 1)) \
        .astype(F32)
    return _dot(oht, scale), _dot(oht, off)


def _norm0_body(g, r, s_ref, stats_ref, batch_ref, dinv_ref,
                gw_ref, gb_ref, gms_ref, w1_ref, res_ref, hs1_ref):
    scale, off = _norm_scale_off(stats_ref[...], gw_ref, gb_ref, gms_ref)
    sc_r, off_r = _row_gather(batch_ref, g, r, scale, off)
    gv = _gelu(s_ref[...] * sc_r + off_r)
    res_ref[...] = gv
    hs1_ref[...] = dinv_ref[0, 0, :][:, None] * _dot(gv, w1_ref[...])


def _tc_norm0(g, np_, nblk, r, s, stats, batch3, dinv, gw, gb, gms, w1big):
    f = pl.pallas_call(
        functools.partial(_norm0_body, g, r),
        grid=(nblk,),
        in_specs=[
            pl.BlockSpec((r, 128), lambda i: (i, 0)),
            pl.BlockSpec((3, g, 128), lambda i: (0, 0, 0)),
            pl.BlockSpec((1, 1, r), lambda i: (i, 0, 0)),
            pl.BlockSpec((1, 1, r), lambda i: (i, 0, 0)),
            pl.BlockSpec((1, 128), lambda i: (0, 0)),
            pl.BlockSpec((1, 128), lambda i: (0, 0)),
            pl.BlockSpec((1, 128), lambda i: (0, 0)),
            pl.BlockSpec((128, 128), lambda i: (0, 0)),
        ],
        out_specs=[
            pl.BlockSpec((r, 128), lambda i: (i, 0)),
            pl.BlockSpec((r, 128), lambda i: (i, 0)),
        ],
        out_shape=[
            jax.ShapeDtypeStruct((np_, 128), F32),
            jax.ShapeDtypeStruct((np_, 128), F32),
        ],
        compiler_params=pltpu.CompilerParams(
            dimension_semantics=("arbitrary",)),
    )
    return f(s, stats, batch3, dinv, gw, gb, gms, w1big)


def _final_body(g, r, s_ref, stats_ref, batch_ref, res_ref,
                gw_ref, gb_ref, gms_ref, hw0_ref, hb0_ref, hw1_ref, hb1_ref,
                z_ref):
    scale, off = _norm_scale_off(stats_ref[...], gw_ref, gb_ref, gms_ref)
    sc_r, off_r = _row_gather(batch_ref, g, r, scale, off)
    h1 = _gelu(s_ref[...] * sc_r + off_r) + res_ref[...]
    t1 = _gelu(_dot(h1, hw0_ref[...]) + hb0_ref[0, :][None, :])
    z_ref[...] = _dot(t1, hw1_ref[...]) + hb1_ref[0, :][None, :]


def _tc_final(g, np_, nblk, r, s, stats, batch3, res0,
              gw, gb, gms, hw0big, hb0t, hw1big, hb1t):
    f = pl.pallas_call(
        functools.partial(_final_body, g, r),
        grid=(nblk,),
        in_specs=[
            pl.BlockSpec((r, 128), lambda i: (i, 0)),
            pl.BlockSpec((3, g, 128), lambda i: (0, 0, 0)),
            pl.BlockSpec((1, 1, r), lambda i: (i, 0, 0)),
            pl.BlockSpec((r, 128), lambda i: (i, 0)),
            pl.BlockSpec((1, 128), lambda i: (0, 0)),
            pl.BlockSpec((1, 128), lambda i: (0, 0)),
            pl.BlockSpec((1, 128), lambda i: (0, 0)),
            pl.BlockSpec((128, 128), lambda i: (0, 0)),
            pl.BlockSpec((1, 128), lambda i: (0, 0)),
            pl.BlockSpec((128, 128), lambda i: (0, 0)),
            pl.BlockSpec((1, 128), lambda i: (0, 0)),
        ],
        out_specs=[pl.BlockSpec((r, 128), lambda i: (i, 0))],
        out_shape=[jax.ShapeDtypeStruct((np_, 128), F32)],
        compiler_params=pltpu.CompilerParams(
            dimension_semantics=("arbitrary",)),
    )
    return f(s, stats, batch3, res0, gw, gb, gms, hw0big, hb0t, hw1big, hb1t)[0]


def kernel(x, batch, edge_index, W0, b0, gn0_w, gn0_b, gn0_ms,
           W1, b1, gn1_w, gn1_b, gn1_ms, hW0, hb0, hW1, hb1):
    n, t, d = x.shape
    h = W0.shape[1]
    e = edge_index.shape[1]
    g = 16
    out_f = hW1.shape[1]

    r = 1280
    np_ = ((n + 1 + r - 1) // r) * r
    nblk = np_ // r
    ncht = (e + NS * CHUNK - 1) // (NS * CHUNK)
    cpt0 = max(GRP, (ncht * 8) // 13 // GRP * GRP)
    cpt1 = max(GRP, (ncht - cpt0 + GRP - 1) // GRP * GRP)
    e_pad = NS * (cpt0 + cpt1) * CHUNK

    x2 = x.reshape(n, t * d)
    xp = jnp.zeros((np_, t * d), F32).at[:n].set(x2)
    batchp = jnp.full((np_,), g, jnp.int32).at[:n].set(batch.astype(jnp.int32))
    batch3 = batchp.reshape(nblk, 1, r)
    ei = edge_index.astype(jnp.int32)
    srcp = jnp.full((e_pad,), n, jnp.int32).at[:e].set(ei[0])
    dstp = jnp.full((e_pad,), n, jnp.int32).at[:e].set(ei[1])

    srci = srcp.reshape(NS * (cpt0 + cpt1), CHUNK)
    dsti = dstp.reshape(NS * (cpt0 + cpt1), CHUNK)

    eye_t = jnp.eye(t, dtype=F32)
    w0big = jnp.kron(eye_t, W0)
    w1big = jnp.kron(eye_t, W1)
    hw0big = jnp.kron(eye_t, hW0)
    hw1big = jnp.zeros((t * h, 128), F32).at[:, :t * out_f].set(
        jnp.kron(eye_t, hW1))
    b0t = jnp.tile(b0, t).reshape(1, t * h)
    b1t = jnp.tile(b1, t).reshape(1, t * h)
    gw0 = jnp.tile(gn0_w, t).reshape(1, t * h)
    gb0 = jnp.tile(gn0_b, t).reshape(1, t * h)
    gm0 = jnp.tile(gn0_ms, t).reshape(1, t * h)
    gw1 = jnp.tile(gn1_w, t).reshape(1, t * h)
    gb1 = jnp.tile(gn1_b, t).reshape(1, t * h)
    gm1 = jnp.tile(gn1_ms, t).reshape(1, t * h)
    hb0t = jnp.tile(hb0, t).reshape(1, t * h)
    hb1t = jnp.zeros((1, 128), F32).at[0, :t * out_f].set(jnp.tile(hb1, t))

    rpw = np_ // NS
    zer128 = jnp.zeros((rpw, 128), F32)
    zer1 = jnp.zeros((640,), F32)
    ones1 = jnp.ones((CHUNK,), F32)

    degp = _sc_degree(cpt0, cpt1, dsti, ones1, zer1)
    deg0 = degp[0, 0, :np_].reshape(nblk, 1, r)
    deg1 = degp[1, 0, :np_].reshape(nblk, 1, r)

    hs0, dinv = _tc_scale(np_, nblk, r, xp, w0big, deg0, deg1)
    parts0 = _sc_aggregate(np_, cpt0, cpt1, hs0, srci, dsti, zer128)
    s0, stats0 = _tc_stats(g, np_, nblk, r, parts0, hs0, dinv, b0t, batch3)
    res0, hs1 = _tc_norm0(g, np_, nblk, r, s0, stats0, batch3, dinv,
                          gw0, gb0, gm0, w1big)
    parts1 = _sc_aggregate(np_, cpt0, cpt1, hs1, srci, dsti, zer128)
    s1, stats1 = _tc_stats(g, np_, nblk, r, parts1, hs1, dinv, b1t, batch3)
    z = _tc_final(g, np_, nblk, r, s1, stats1, batch3, res0,
                  gw1, gb1, gm1, hw0big, hb0t, hw1big, hb1t)

    return z[:n, :t * out_f].reshape(n, t, out_f)


# speedup vs baseline: 1.7316x; 1.7316x over previous
import functools

import jax
import jax.numpy as jnp
from jax import lax
from jax.experimental import pallas as pl
from jax.experimental.pallas import tpu as pltpu
from jax.experimental.pallas import tpu_sc as plsc

NC = 2
NS = 16
NW = NC * NS
CHUNK = 128
EPS = 1e-5

F32 = jnp.float32


def _sc_mesh():
    return plsc.VectorSubcoreMesh(
        core_axis_name="c", subcore_axis_name="s",
        num_cores=NC, num_subcores=NS)


NPD = NS * 640


def _deg_body(cpt0, cpt1, dsti, ones_hbm, zer_hbm, out, dst_v, ones_v, dacc):
    c = lax.axis_index("c")
    s = lax.axis_index("s")
    nch = jnp.where(c == 0, cpt0, cpt1)
    pltpu.sync_copy(zer_hbm, dacc.at[pl.ds(s * 640, 640)])
    pltpu.sync_copy(ones_hbm, ones_v)

    @pl.when(c == 0)
    def _():
        off = pl.multiple_of(s * cpt0, GRP)
        pltpu.sync_copy(dsti.at[pl.ds(off, cpt0)], dst_v.at[pl.ds(0, cpt0)])

    @pl.when(c != 0)
    def _():
        off = pl.multiple_of(NS * cpt0 + s * cpt1, GRP)
        pltpu.sync_copy(dsti.at[pl.ds(off, cpt1)], dst_v.at[pl.ds(0, cpt1)])

    plsc.subcore_barrier()

    def body(ci, carry):
        pltpu.sync_copy(ones_v, dacc.at[dst_v.at[ci]], add=True)
        return carry

    lax.fori_loop(0, nch, body, 0)
    plsc.subcore_barrier()
    pltpu.sync_copy(dacc.at[pl.ds(s * 640, 640)],
                    out.at[c, 0, pl.ds(s * 640, 640)])


def _sc_degree(cpt0, cpt1, dsti, ones_in, zer_in):
    f = pl.kernel(
        functools.partial(_deg_body, cpt0, cpt1),
        out_type=jax.ShapeDtypeStruct((NC, 1, NPD), F32),
        mesh=_sc_mesh(),
        scratch_types=[
            pltpu.VMEM((cpt0, CHUNK), jnp.int32),
            pltpu.VMEM((CHUNK,), F32),
            pltpu.VMEM_SHARED((NPD,), F32),
        ],
    )
    return f(dsti, ones_in, zer_in)


GRP = 8


def _agg_body(cpt0, cpt1, rpw, hs, srci, dsti, zer_hbm, out,
              src_v, dst_v, rows0, acc, sem0):
    c = lax.axis_index("c")
    s = lax.axis_index("s")
    nch = jnp.where(c == 0, cpt0, cpt1)
    pltpu.sync_copy(zer_hbm, acc.at[pl.ds(s * rpw, rpw)])

    @pl.when(c == 0)
    def _():
        off = pl.multiple_of(s * cpt0, GRP)
        pltpu.sync_copy(srci.at[pl.ds(off, cpt0)], src_v.at[pl.ds(0, cpt0)])
        pltpu.sync_copy(dsti.at[pl.ds(off, cpt0)], dst_v.at[pl.ds(0, cpt0)])

    @pl.when(c != 0)
    def _():
        off = pl.multiple_of(NS * cpt0 + s * cpt1, GRP)
        pltpu.sync_copy(srci.at[pl.ds(off, cpt1)], src_v.at[pl.ds(0, cpt1)])
        pltpu.sync_copy(dsti.at[pl.ds(off, cpt1)], dst_v.at[pl.ds(0, cpt1)])

    plsc.subcore_barrier()

    def body(i, carry):
        pltpu.async_copy(hs.at[src_v.at[i]], rows0, sem0).wait()
        pltpu.sync_copy(rows0, acc.at[dst_v.at[i]], add=True)
        return carry

    lax.fori_loop(0, nch, body, 0)
    plsc.subcore_barrier()
    pltpu.sync_copy(acc.at[pl.ds(s * rpw, rpw)],
                    out.at[c, pl.ds(s * rpw, rpw)])


def _sc_aggregate(np_, cpt0, cpt1, hs, srci, dsti, zer_in):
    rpw = np_ // NS
    f = pl.kernel(
        functools.partial(_agg_body, cpt0, cpt1, rpw),
        out_type=jax.ShapeDtypeStruct((NC, np_, 128), F32),
        mesh=_sc_mesh(),
        scratch_types=[
            pltpu.VMEM((cpt0, CHUNK), jnp.int32),
            pltpu.VMEM((cpt0, CHUNK), jnp.int32),
            pltpu.VMEM((CHUNK, 128), F32),
            pltpu.VMEM_SHARED((np_, 128), F32),
            pltpu.SemaphoreType.DMA,
        ],
    )
    return f(hs, srci, dsti, zer_in)


def _gelu(v):
    return 0.5 * v * (1.0 + lax.erf(v * (2.0 ** -0.5)))


def _dot(a, b):
    return jnp.dot(a, b, preferred_element_type=F32,
                   precision=lax.Precision.HIGHEST)


def _scale_body(x_ref, w_ref, d0_ref, d1_ref, hs_ref, dinv_ref):
    deg = d0_ref[0, 0, :] + d1_ref[0, 0, :] + 1.0
    dv = lax.rsqrt(deg)
    dinv_ref[0, 0, :] = dv
    hs_ref[...] = dv[:, None] * _dot(x_ref[...], w_ref[...])


def _tc_scale(np_, nblk, r, xp, w0big, deg0, deg1):
    f = pl.pallas_call(
        _scale_body,
        grid=(nblk,),
        in_specs=[
            pl.BlockSpec((r, 512), lambda i: (i, 0)),
            pl.BlockSpec((512, 128), lambda i: (0, 0)),
            pl.BlockSpec((1, 1, r), lambda i: (i, 0, 0)),
            pl.BlockSpec((1, 1, r), lambda i: (i, 0, 0)),
        ],
        out_specs=[
            pl.BlockSpec((r, 128), lambda i: (i, 0)),
            pl.BlockSpec((1, 1, r), lambda i: (i, 0, 0)),
        ],
        out_shape=[
            jax.ShapeDtypeStruct((np_, 128), F32),
            jax.ShapeDtypeStruct((nblk, 1, r), F32),
        ],
        compiler_params=pltpu.CompilerParams(
            dimension_semantics=("arbitrary",)),
    )
    return f(xp, w0big, deg0, deg1)


def _stats_body(g, r, parts_ref, hs_ref, dinv_ref, bt_ref, batch_ref,
                s_ref, stats_ref):
    i = pl.program_id(0)
    dv = dinv_ref[0, 0, :]
    sv = dv[:, None] * (parts_ref[0] + parts_ref[1] + hs_ref[...]) \
        + bt_ref[0, :][None, :]
    s_ref[...] = sv
    b = batch_ref[0, 0, :]
    oh = (lax.broadcasted_iota(jnp.int32, (g, r), 0) == b[None, :]) \
        .astype(F32)
    a1 = _dot(oh, sv)
    a2 = _dot(oh, sv * sv)
    cnt = jnp.sum(oh, axis=1)
    new = jnp.stack([a1, a2, jnp.broadcast_to(cnt[:, None], (g, 128))])

    @pl.when(i == 0)
    def _():
        stats_ref[...] = new

    @pl.when(i > 0)
    def _():
        stats_ref[...] += new


def _tc_stats(g, np_, nblk, r, parts, hs, dinv, bt, batch3):
    f = pl.pallas_call(
        functools.partial(_stats_body, g, r),
        grid=(nblk,),
        in_specs=[
            pl.BlockSpec((2, r, 128), lambda i: (0, i, 0)),
            pl.BlockSpec((r, 128), lambda i: (i, 0)),
            pl.BlockSpec((1, 1, r), lambda i: (i, 0, 0)),
            pl.BlockSpec((1, 128), lambda i: (0, 0)),
            pl.BlockSpec((1, 1, r), lambda i: (i, 0, 0)),
        ],
        out_specs=[
            pl.BlockSpec((r, 128), lambda i: (i, 0)),
            pl.BlockSpec((3, g, 128), lambda i: (0, 0, 0)),
        ],
        out_shape=[
            jax.ShapeDtypeStruct((np_, 128), F32),
            jax.ShapeDtypeStruct((3, g, 128), F32),
        ],
        compiler_params=pltpu.CompilerParams(
            dimension_semantics=("arbitrary",)),
    )
    return f(parts, hs, dinv, bt, batch3)


def _norm_scale_off(stats, gw, gb, gms):
    cnt = jnp.maximum(stats[2], 1.0)
    mean = stats[0] / cnt
    e2 = stats[1] / cnt
    ms = gms[0, :][None, :]
    var = e2 - mean * mean * ms * (2.0 - ms)
    rs = lax.rsqrt(var + EPS)
    w = gw[0, :][None, :]
    scale = w * rs
    off = gb[0, :][None, :] - w * ms * mean * rs
    return scale, off


def _row_gather(batch_ref, g, r, scale, off):
    b = batch_ref[0, 0, :]
    oht = (b[:, None] == lax.broadcasted_iota(jnp.int32, (r, g), 1)) \
        .astype(F32)
    return _dot(oht, scale), _dot(oht, off)


def _norm0_body(g, r, s_ref, stats_ref, batch_ref, dinv_ref,
                gw_ref, gb_ref, gms_ref, w1_ref, res_ref, hs1_ref):
    scale, off = _norm_scale_off(stats_ref[...], gw_ref, gb_ref, gms_ref)
    sc_r, off_r = _row_gather(batch_ref, g, r, scale, off)
    gv = _gelu(s_ref[...] * sc_r + off_r)
    res_ref[...] = gv
    hs1_ref[...] = dinv_ref[0, 0, :][:, None] * _dot(gv, w1_ref[...])


def _tc_norm0(g, np_, nblk, r, s, stats, batch3, dinv, gw, gb, gms, w1big):
    f = pl.pallas_call(
        functools.partial(_norm0_body, g, r),
        grid=(nblk,),
        in_specs=[
            pl.BlockSpec((r, 128), lambda i: (i, 0)),
            pl.BlockSpec((3, g, 128), lambda i: (0, 0, 0)),
            pl.BlockSpec((1, 1, r), lambda i: (i, 0, 0)),
            pl.BlockSpec((1, 1, r), lambda i: (i, 0, 0)),
            pl.BlockSpec((1, 128), lambda i: (0, 0)),
            pl.BlockSpec((1, 128), lambda i: (0, 0)),
            pl.BlockSpec((1, 128), lambda i: (0, 0)),
            pl.BlockSpec((128, 128), lambda i: (0, 0)),
        ],
        out_specs=[
            pl.BlockSpec((r, 128), lambda i: (i, 0)),
            pl.BlockSpec((r, 128), lambda i: (i, 0)),
        ],
        out_shape=[
            jax.ShapeDtypeStruct((np_, 128), F32),
            jax.ShapeDtypeStruct((np_, 128), F32),
        ],
        compiler_params=pltpu.CompilerParams(
            dimension_semantics=("arbitrary",)),
    )
    return f(s, stats, batch3, dinv, gw, gb, gms, w1big)


def _final_body(g, r, s_ref, stats_ref, batch_ref, res_ref,
                gw_ref, gb_ref, gms_ref, hw0_ref, hb0_ref, hw1_ref, hb1_ref,
                z_ref):
    scale, off = _norm_scale_off(stats_ref[...], gw_ref, gb_ref, gms_ref)
    sc_r, off_r = _row_gather(batch_ref, g, r, scale, off)
    h1 = _gelu(s_ref[...] * sc_r + off_r) + res_ref[...]
    t1 = _gelu(_dot(h1, hw0_ref[...]) + hb0_ref[0, :][None, :])
    z_ref[...] = _dot(t1, hw1_ref[...]) + hb1_ref[0, :][None, :]


def _tc_final(g, np_, nblk, r, s, stats, batch3, res0,
              gw, gb, gms, hw0big, hb0t, hw1big, hb1t):
    f = pl.pallas_call(
        functools.partial(_final_body, g, r),
        grid=(nblk,),
        in_specs=[
            pl.BlockSpec((r, 128), lambda i: (i, 0)),
            pl.BlockSpec((3, g, 128), lambda i: (0, 0, 0)),
            pl.BlockSpec((1, 1, r), lambda i: (i, 0, 0)),
            pl.BlockSpec((r, 128), lambda i: (i, 0)),
            pl.BlockSpec((1, 128), lambda i: (0, 0)),
            pl.BlockSpec((1, 128), lambda i: (0, 0)),
            pl.BlockSpec((1, 128), lambda i: (0, 0)),
            pl.BlockSpec((128, 128), lambda i: (0, 0)),
            pl.BlockSpec((1, 128), lambda i: (0, 0)),
            pl.BlockSpec((128, 128), lambda i: (0, 0)),
            pl.BlockSpec((1, 128), lambda i: (0, 0)),
        ],
        out_specs=[pl.BlockSpec((r, 128), lambda i: (i, 0))],
        out_shape=[jax.ShapeDtypeStruct((np_, 128), F32)],
        compiler_params=pltpu.CompilerParams(
            dimension_semantics=("arbitrary",)),
    )
    return f(s, stats, batch3, res0, gw, gb, gms, hw0big, hb0t, hw1big, hb1t)[0]


def kernel(x, batch, edge_index, W0, b0, gn0_w, gn0_b, gn0_ms,
           W1, b1, gn1_w, gn1_b, gn1_ms, hW0, hb0, hW1, hb1):
    n, t, d = x.shape
    h = W0.shape[1]
    e = edge_index.shape[1]
    g = 16
    out_f = hW1.shape[1]

    r = 1280
    np_ = ((n + 1 + r - 1) // r) * r
    nblk = np_ // r
    ncht = (e + NS * CHUNK - 1) // (NS * CHUNK)
    cpt0 = max(GRP, (ncht * 8) // 13 // GRP * GRP)
    cpt1 = max(GRP, (ncht - cpt0 + GRP - 1) // GRP * GRP)
    e_pad = NS * (cpt0 + cpt1) * CHUNK

    x2 = x.reshape(n, t * d)
    xp = jnp.zeros((np_, t * d), F32).at[:n].set(x2)
    batchp = jnp.full((np_,), g, jnp.int32).at[:n].set(batch.astype(jnp.int32))
    batch3 = batchp.reshape(nblk, 1, r)
    ei = edge_index.astype(jnp.int32)
    pad_idx = n + jnp.arange(e_pad - e, dtype=jnp.int32) % (np_ - n)
    srcp = jnp.concatenate([ei[0], pad_idx])
    dstp = jnp.concatenate([ei[1], pad_idx])

    srci = srcp.reshape(NS * (cpt0 + cpt1), CHUNK)
    dsti = dstp.reshape(NS * (cpt0 + cpt1), CHUNK)

    eye_t = jnp.eye(t, dtype=F32)
    w0big = jnp.kron(eye_t, W0)
    w1big = jnp.kron(eye_t, W1)
    hw0big = jnp.kron(eye_t, hW0)
    hw1big = jnp.zeros((t * h, 128), F32).at[:, :t * out_f].set(
        jnp.kron(eye_t, hW1))
    b0t = jnp.tile(b0, t).reshape(1, t * h)
    b1t = jnp.tile(b1, t).reshape(1, t * h)
    gw0 = jnp.tile(gn0_w, t).reshape(1, t * h)
    gb0 = jnp.tile(gn0_b, t).reshape(1, t * h)
    gm0 = jnp.tile(gn0_ms, t).reshape(1, t * h)
    gw1 = jnp.tile(gn1_w, t).reshape(1, t * h)
    gb1 = jnp.tile(gn1_b, t).reshape(1, t * h)
    gm1 = jnp.tile(gn1_ms, t).reshape(1, t * h)
    hb0t = jnp.tile(hb0, t).reshape(1, t * h)
    hb1t = jnp.zeros((1, 128), F32).at[0, :t * out_f].set(jnp.tile(hb1, t))

    rpw = np_ // NS
    zer128 = jnp.zeros((rpw, 128), F32)
    zer1 = jnp.zeros((640,), F32)
    ones1 = jnp.ones((CHUNK,), F32)

    degp = _sc_degree(cpt0, cpt1, dsti, ones1, zer1)
    deg0 = degp[0, 0, :np_].reshape(nblk, 1, r)
    deg1 = degp[1, 0, :np_].reshape(nblk, 1, r)

    hs0, dinv = _tc_scale(np_, nblk, r, xp, w0big, deg0, deg1)
    parts0 = _sc_aggregate(np_, cpt0, cpt1, hs0, srci, dsti, zer128)
    s0, stats0 = _tc_stats(g, np_, nblk, r, parts0, hs0, dinv, b0t, batch3)
    res0, hs1 = _tc_norm0(g, np_, nblk, r, s0, stats0, batch3, dinv,
                          gw0, gb0, gm0, w1big)
    parts1 = _sc_aggregate(np_, cpt0, cpt1, hs1, srci, dsti, zer128)
    s1, stats1 = _tc_stats(g, np_, nblk, r, parts1, hs1, dinv, b1t, batch3)
    z = _tc_final(g, np_, nblk, r, s1, stats1, batch3, res0,
                  gw1, gb1, gm1, hw0big, hb0t, hw1big, hb1t)

    return z[:n, :t * out_f].reshape(n, t, out_f)


# speedup vs baseline: 1.9543x; 1.1286x over previous
import functools

import jax
import jax.numpy as jnp
from jax import lax
from jax.experimental import pallas as pl
from jax.experimental.pallas import tpu as pltpu
from jax.experimental.pallas import tpu_sc as plsc

NC = 2
NS = 16
NW = NC * NS
CHUNK = 128
EPS = 1e-5

F32 = jnp.float32


def _sc_mesh():
    return plsc.VectorSubcoreMesh(
        core_axis_name="c", subcore_axis_name="s",
        num_cores=NC, num_subcores=NS)


NPD = NS * 640


def _deg_body(cpt0, cpt1, dsti, ones_hbm, zer_hbm, out, dst_v, ones_v, dacc):
    c = lax.axis_index("c")
    s = lax.axis_index("s")
    nch = jnp.where(c == 0, cpt0, cpt1)
    pltpu.sync_copy(zer_hbm, dacc.at[pl.ds(s * 640, 640)])
    pltpu.sync_copy(ones_hbm, ones_v)

    @pl.when(c == 0)
    def _():
        off = pl.multiple_of(s * cpt0, GRP)
        pltpu.sync_copy(dsti.at[pl.ds(off, cpt0)], dst_v.at[pl.ds(0, cpt0)])

    @pl.when(c != 0)
    def _():
        off = pl.multiple_of(NS * cpt0 + s * cpt1, GRP)
        pltpu.sync_copy(dsti.at[pl.ds(off, cpt1)], dst_v.at[pl.ds(0, cpt1)])

    plsc.subcore_barrier()

    def body(ci, carry):
        pltpu.sync_copy(ones_v, dacc.at[dst_v.at[ci]], add=True)
        return carry

    lax.fori_loop(0, nch, body, 0)
    plsc.subcore_barrier()
    pltpu.sync_copy(dacc.at[pl.ds(s * 640, 640)],
                    out.at[c, 0, pl.ds(s * 640, 640)])


def _sc_degree(cpt0, cpt1, dsti, ones_in, zer_in):
    f = pl.kernel(
        functools.partial(_deg_body, cpt0, cpt1),
        out_type=jax.ShapeDtypeStruct((NC, 1, NPD), F32),
        mesh=_sc_mesh(),
        scratch_types=[
            pltpu.VMEM((cpt0, CHUNK), jnp.int32),
            pltpu.VMEM((CHUNK,), F32),
            pltpu.VMEM_SHARED((NPD,), F32),
        ],
    )
    return f(dsti, ones_in, zer_in)


GRP = 8


def _agg_body(cpt0, cpt1, rpw, hs, srci, dsti, zer_hbm, out,
              sring, dring, rows0, rows1, acc, semg0, semg1, sems, semd):
    c = lax.axis_index("c")
    s = lax.axis_index("s")
    ngrp = jnp.where(c == 0, cpt0 // GRP, cpt1 // GRP)
    row0 = jnp.where(c == 0, s * cpt0, NS * cpt0 + s * cpt1)
    row0 = pl.multiple_of(row0, GRP)
    pltpu.sync_copy(zer_hbm, acc.at[pl.ds(s * rpw, rpw)])
    plsc.subcore_barrier()

    pltpu.sync_copy(srci.at[pl.ds(row0, GRP)], sring.at[pl.ds(0, GRP)])
    pltpu.sync_copy(dsti.at[pl.ds(row0, GRP)], dring.at[pl.ds(0, GRP)])
    pltpu.async_copy(hs.at[sring.at[0]], rows0, semg0)

    def _ring_prefetch(g):
        q = lax.rem(g + 1, 2)
        src_off = pl.multiple_of(row0 + (g + 1) * GRP, GRP)
        dst_off = pl.multiple_of(q * GRP, GRP)
        pltpu.async_copy(srci.at[pl.ds(src_off, GRP)],
                         sring.at[pl.ds(dst_off, GRP)], sems)
        pltpu.async_copy(dsti.at[pl.ds(src_off, GRP)],
                         dring.at[pl.ds(dst_off, GRP)], semd)

    def _ring_wait():
        pltpu.make_async_copy(srci.at[pl.ds(0, GRP)],
                              sring.at[pl.ds(0, GRP)], sems).wait()
        pltpu.make_async_copy(dsti.at[pl.ds(0, GRP)],
                              dring.at[pl.ds(0, GRP)], semd).wait()

    def body0(g, carry):
        p = lax.rem(g, 2)
        q = 1 - p

        @pl.when(g + 1 < ngrp)
        def _():
            _ring_prefetch(g)

        for k in range(GRP):
            buf, sem = (rows0, semg0) if k % 2 == 0 else (rows1, semg1)
            nbuf, nsem = (rows1, semg1) if k % 2 == 0 else (rows0, semg0)
            pltpu.make_async_copy(hs.at[sring.at[0]], buf, sem).wait()
            if k == GRP - 1:
                @pl.when(g + 1 < ngrp)
                def _():
                    _ring_wait()
                    pltpu.async_copy(hs.at[sring.at[q * GRP]], nbuf, nsem)
            else:
                pltpu.async_copy(hs.at[sring.at[p * GRP + k + 1]], nbuf, nsem)
            pltpu.sync_copy(buf, acc.at[dring.at[p * GRP + k]], add=True)
        return carry

    def body1(g, carry):
        p = lax.rem(g, 2)
        q = 1 - p

        @pl.when(g + 1 < ngrp)
        def _():
            _ring_prefetch(g)

        for k in range(GRP):
            pltpu.make_async_copy(hs.at[sring.at[0]], rows0, semg0).wait()
            pltpu.sync_copy(rows0, acc.at[dring.at[p * GRP + k]], add=True)
            if k == GRP - 1:
                @pl.when(g + 1 < ngrp)
                def _():
                    _ring_wait()
                    pltpu.async_copy(hs.at[sring.at[q * GRP]], rows0, semg0)
            else:
                pltpu.async_copy(hs.at[sring.at[p * GRP + k + 1]], rows0,
                                 semg0)
        return carry

    @pl.when(c == 0)
    def _():
        lax.fori_loop(0, ngrp, body0, 0)

    @pl.when(c != 0)
    def _():
        lax.fori_loop(0, ngrp, body1, 0)

    plsc.subcore_barrier()
    pltpu.sync_copy(acc.at[pl.ds(s * rpw, rpw)],
                    out.at[c, pl.ds(s * rpw, rpw)])


def _sc_aggregate(np_, cpt0, cpt1, hs, srci, dsti, zer_in):
    rpw = np_ // NS
    f = pl.kernel(
        functools.partial(_agg_body, cpt0, cpt1, rpw),
        out_type=jax.ShapeDtypeStruct((NC, np_, 128), F32),
        mesh=_sc_mesh(),
        scratch_types=[
            pltpu.VMEM((2 * GRP, CHUNK), jnp.int32),
            pltpu.VMEM((2 * GRP, CHUNK), jnp.int32),
            pltpu.VMEM((CHUNK, 128), F32),
            pltpu.VMEM((CHUNK, 128), F32),
            pltpu.VMEM_SHARED((np_, 128), F32),
            pltpu.SemaphoreType.DMA,
            pltpu.SemaphoreType.DMA,
            pltpu.SemaphoreType.DMA,
            pltpu.SemaphoreType.DMA,
        ],
    )
    return f(hs, srci, dsti, zer_in)


def _gelu(v):
    return 0.5 * v * (1.0 + lax.erf(v * (2.0 ** -0.5)))


def _dot(a, b):
    return jnp.dot(a, b, preferred_element_type=F32,
                   precision=lax.Precision.HIGHEST)


def _scale_body(x_ref, w_ref, d0_ref, d1_ref, hs_ref, dinv_ref):
    deg = d0_ref[0, 0, :] + d1_ref[0, 0, :] + 1.0
    dv = lax.rsqrt(deg)
    dinv_ref[0, 0, :] = dv
    hs_ref[...] = dv[:, None] * _dot(x_ref[...], w_ref[...])


def _tc_scale(np_, nblk, r, xp, w0big, deg0, deg1):
    f = pl.pallas_call(
        _scale_body,
        grid=(nblk,),
        in_specs=[
            pl.BlockSpec((r, 512), lambda i: (i, 0)),
            pl.BlockSpec((512, 128), lambda i: (0, 0)),
            pl.BlockSpec((1, 1, r), lambda i: (i, 0, 0)),
            pl.BlockSpec((1, 1, r), lambda i: (i, 0, 0)),
        ],
        out_specs=[
            pl.BlockSpec((r, 128), lambda i: (i, 0)),
            pl.BlockSpec((1, 1, r), lambda i: (i, 0, 0)),
        ],
        out_shape=[
            jax.ShapeDtypeStruct((np_, 128), F32),
            jax.ShapeDtypeStruct((nblk, 1, r), F32),
        ],
        compiler_params=pltpu.CompilerParams(
            dimension_semantics=("arbitrary",)),
    )
    return f(xp, w0big, deg0, deg1)


def _stats_body(g, r, parts_ref, hs_ref, dinv_ref, bt_ref, batch_ref,
                s_ref, stats_ref):
    i = pl.program_id(0)
    dv = dinv_ref[0, 0, :]
    sv = dv[:, None] * (parts_ref[0] + parts_ref[1] + hs_ref[...]) \
        + bt_ref[0, :][None, :]
    s_ref[...] = sv
    b = batch_ref[0, 0, :]
    oh = (lax.broadcasted_iota(jnp.int32, (g, r), 0) == b[None, :]) \
        .astype(F32)
    a1 = _dot(oh, sv)
    a2 = _dot(oh, sv * sv)
    cnt = jnp.sum(oh, axis=1)
    new = jnp.stack([a1, a2, jnp.broadcast_to(cnt[:, None], (g, 128))])

    @pl.when(i == 0)
    def _():
        stats_ref[...] = new

    @pl.when(i > 0)
    def _():
        stats_ref[...] += new


def _tc_stats(g, np_, nblk, r, parts, hs, dinv, bt, batch3):
    f = pl.pallas_call(
        functools.partial(_stats_body, g, r),
        grid=(nblk,),
        in_specs=[
            pl.BlockSpec((2, r, 128), lambda i: (0, i, 0)),
            pl.BlockSpec((r, 128), lambda i: (i, 0)),
            pl.BlockSpec((1, 1, r), lambda i: (i, 0, 0)),
            pl.BlockSpec((1, 128), lambda i: (0, 0)),
            pl.BlockSpec((1, 1, r), lambda i: (i, 0, 0)),
        ],
        out_specs=[
            pl.BlockSpec((r, 128), lambda i: (i, 0)),
            pl.BlockSpec((3, g, 128), lambda i: (0, 0, 0)),
        ],
        out_shape=[
            jax.ShapeDtypeStruct((np_, 128), F32),
            jax.ShapeDtypeStruct((3, g, 128), F32),
        ],
        compiler_params=pltpu.CompilerParams(
            dimension_semantics=("arbitrary",)),
    )
    return f(parts, hs, dinv, bt, batch3)


def _norm_scale_off(stats, gw, gb, gms):
    cnt = jnp.maximum(stats[2], 1.0)
    mean = stats[0] / cnt
    e2 = stats[1] / cnt
    ms = gms[0, :][None, :]
    var = e2 - mean * mean * ms * (2.0 - ms)
    rs = lax.rsqrt(var + EPS)
    w = gw[0, :][None, :]
    scale = w * rs
    off = gb[0, :][None, :] - w * ms * mean * rs
    return scale, off


def _row_gather(batch_ref, g, r, scale, off):
    b = batch_ref[0, 0, :]
    oht = (b[:, None] == lax.broadcasted_iota(jnp.int32, (r, g), 1)) \
        .astype(F32)
    return _dot(oht, scale), _dot(oht, off)


def _norm0_body(g, r, s_ref, stats_ref, batch_ref, dinv_ref,
                gw_ref, gb_ref, gms_ref, w1_ref, res_ref, hs1_ref):
    scale, off = _norm_scale_off(stats_ref[...], gw_ref, gb_ref, gms_ref)
    sc_r, off_r = _row_gather(batch_ref, g, r, scale, off)
    gv = _gelu(s_ref[...] * sc_r + off_r)
    res_ref[...] = gv
    hs1_ref[...] = dinv_ref[0, 0, :][:, None] * _dot(gv, w1_ref[...])


def _tc_norm0(g, np_, nblk, r, s, stats, batch3, dinv, gw, gb, gms, w1big):
    f = pl.pallas_call(
        functools.partial(_norm0_body, g, r),
        grid=(nblk,),
        in_specs=[
            pl.BlockSpec((r, 128), lambda i: (i, 0)),
            pl.BlockSpec((3, g, 128), lambda i: (0, 0, 0)),
            pl.BlockSpec((1, 1, r), lambda i: (i, 0, 0)),
            pl.BlockSpec((1, 1, r), lambda i: (i, 0, 0)),
            pl.BlockSpec((1, 128), lambda i: (0, 0)),
            pl.BlockSpec((1, 128), lambda i: (0, 0)),
            pl.BlockSpec((1, 128), lambda i: (0, 0)),
            pl.BlockSpec((128, 128), lambda i: (0, 0)),
        ],
        out_specs=[
            pl.BlockSpec((r, 128), lambda i: (i, 0)),
            pl.BlockSpec((r, 128), lambda i: (i, 0)),
        ],
        out_shape=[
            jax.ShapeDtypeStruct((np_, 128), F32),
            jax.ShapeDtypeStruct((np_, 128), F32),
        ],
        compiler_params=pltpu.CompilerParams(
            dimension_semantics=("arbitrary",)),
    )
    return f(s, stats, batch3, dinv, gw, gb, gms, w1big)


def _final_body(g, r, s_ref, stats_ref, batch_ref, res_ref,
                gw_ref, gb_ref, gms_ref, hw0_ref, hb0_ref, hw1_ref, hb1_ref,
                z_ref):
    scale, off = _norm_scale_off(stats_ref[...], gw_ref, gb_ref, gms_ref)
    sc_r, off_r = _row_gather(batch_ref, g, r, scale, off)
    h1 = _gelu(s_ref[...] * sc_r + off_r) + res_ref[...]
    t1 = _gelu(_dot(h1, hw0_ref[...]) + hb0_ref[0, :][None, :])
    z_ref[...] = _dot(t1, hw1_ref[...]) + hb1_ref[0, :][None, :]


def _tc_final(g, np_, nblk, r, s, stats, batch3, res0,
              gw, gb, gms, hw0big, hb0t, hw1big, hb1t):
    f = pl.pallas_call(
        functools.partial(_final_body, g, r),
        grid=(nblk,),
        in_specs=[
            pl.BlockSpec((r, 128), lambda i: (i, 0)),
            pl.BlockSpec((3, g, 128), lambda i: (0, 0, 0)),
            pl.BlockSpec((1, 1, r), lambda i: (i, 0, 0)),
            pl.BlockSpec((r, 128), lambda i: (i, 0)),
            pl.BlockSpec((1, 128), lambda i: (0, 0)),
            pl.BlockSpec((1, 128), lambda i: (0, 0)),
            pl.BlockSpec((1, 128), lambda i: (0, 0)),
            pl.BlockSpec((128, 128), lambda i: (0, 0)),
            pl.BlockSpec((1, 128), lambda i: (0, 0)),
            pl.BlockSpec((128, 128), lambda i: (0, 0)),
            pl.BlockSpec((1, 128), lambda i: (0, 0)),
        ],
        out_specs=[pl.BlockSpec((r, 128), lambda i: (i, 0))],
        out_shape=[jax.ShapeDtypeStruct((np_, 128), F32)],
        compiler_params=pltpu.CompilerParams(
            dimension_semantics=("arbitrary",)),
    )
    return f(s, stats, batch3, res0, gw, gb, gms, hw0big, hb0t, hw1big, hb1t)[0]


def kernel(x, batch, edge_index, W0, b0, gn0_w, gn0_b, gn0_ms,
           W1, b1, gn1_w, gn1_b, gn1_ms, hW0, hb0, hW1, hb1):
    n, t, d = x.shape
    h = W0.shape[1]
    e = edge_index.shape[1]
    g = 16
    out_f = hW1.shape[1]

    r = 1280
    np_ = ((n + 1 + r - 1) // r) * r
    nblk = np_ // r
    ncht = (e + NS * CHUNK - 1) // (NS * CHUNK)
    cpt0 = max(GRP, (ncht * 5) // 7 // GRP * GRP)
    cpt1 = max(GRP, (ncht - cpt0 + GRP - 1) // GRP * GRP)
    e_pad = NS * (cpt0 + cpt1) * CHUNK

    x2 = x.reshape(n, t * d)
    xp = jnp.zeros((np_, t * d), F32).at[:n].set(x2)
    batchp = jnp.full((np_,), g, jnp.int32).at[:n].set(batch.astype(jnp.int32))
    batch3 = batchp.reshape(nblk, 1, r)
    ei = edge_index.astype(jnp.int32)
    pad_idx = n + jnp.arange(e_pad - e, dtype=jnp.int32) % (np_ - n)
    srcp = jnp.concatenate([ei[0], pad_idx])
    dstp = jnp.concatenate([ei[1], pad_idx])

    srci = srcp.reshape(NS * (cpt0 + cpt1), CHUNK)
    dsti = dstp.reshape(NS * (cpt0 + cpt1), CHUNK)

    eye_t = jnp.eye(t, dtype=F32)
    w0big = jnp.kron(eye_t, W0)
    w1big = jnp.kron(eye_t, W1)
    hw0big = jnp.kron(eye_t, hW0)
    hw1big = jnp.zeros((t * h, 128), F32).at[:, :t * out_f].set(
        jnp.kron(eye_t, hW1))
    b0t = jnp.tile(b0, t).reshape(1, t * h)
    b1t = jnp.tile(b1, t).reshape(1, t * h)
    gw0 = jnp.tile(gn0_w, t).reshape(1, t * h)
    gb0 = jnp.tile(gn0_b, t).reshape(1, t * h)
    gm0 = jnp.tile(gn0_ms, t).reshape(1, t * h)
    gw1 = jnp.tile(gn1_w, t).reshape(1, t * h)
    gb1 = jnp.tile(gn1_b, t).reshape(1, t * h)
    gm1 = jnp.tile(gn1_ms, t).reshape(1, t * h)
    hb0t = jnp.tile(hb0, t).reshape(1, t * h)
    hb1t = jnp.zeros((1, 128), F32).at[0, :t * out_f].set(jnp.tile(hb1, t))

    rpw = np_ // NS
    zer128 = jnp.zeros((rpw, 128), F32)
    zer1 = jnp.zeros((640,), F32)
    ones1 = jnp.ones((CHUNK,), F32)

    degp = _sc_degree(cpt0, cpt1, dsti, ones1, zer1)
    deg0 = degp[0, 0, :np_].reshape(nblk, 1, r)
    deg1 = degp[1, 0, :np_].reshape(nblk, 1, r)

    hs0, dinv = _tc_scale(np_, nblk, r, xp, w0big, deg0, deg1)
    parts0 = _sc_aggregate(np_, cpt0, cpt1, hs0, srci, dsti, zer128)
    s0, stats0 = _tc_stats(g, np_, nblk, r, parts0, hs0, dinv, b0t, batch3)
    res0, hs1 = _tc_norm0(g, np_, nblk, r, s0, stats0, batch3, dinv,
                          gw0, gb0, gm0, w1big)
    parts1 = _sc_aggregate(np_, cpt0, cpt1, hs1, srci, dsti, zer128)
    s1, stats1 = _tc_stats(g, np_, nblk, r, parts1, hs1, dinv, b1t, batch3)
    z = _tc_final(g, np_, nblk, r, s1, stats1, batch3, res0,
                  gw1, gb1, gm1, hw0big, hb0t, hw1big, hb1t)

    return z[:n, :t * out_f].reshape(n, t, out_f)


# speedup vs baseline: 2.1508x; 1.1005x over previous
import functools

import jax
import jax.numpy as jnp
from jax import lax
from jax.experimental import pallas as pl
from jax.experimental.pallas import tpu as pltpu
from jax.experimental.pallas import tpu_sc as plsc

NC = 2
NS = 16
NW = NC * NS
CHUNK = 128
EPS = 1e-5

F32 = jnp.float32


def _sc_mesh():
    return plsc.VectorSubcoreMesh(
        core_axis_name="c", subcore_axis_name="s",
        num_cores=NC, num_subcores=NS)


NPD = NS * 640


def _deg_body(cpt0, cpt1, dsti, ones_hbm, zer_hbm, out, dst_v, ones_v, dacc):
    c = lax.axis_index("c")
    s = lax.axis_index("s")
    nch = jnp.where(c == 0, cpt0, cpt1)
    pltpu.sync_copy(zer_hbm, dacc.at[pl.ds(s * 640, 640)])
    pltpu.sync_copy(ones_hbm, ones_v)

    @pl.when(c == 0)
    def _():
        off = pl.multiple_of(s * cpt0, GRP)
        pltpu.sync_copy(dsti.at[pl.ds(off, cpt0)], dst_v.at[pl.ds(0, cpt0)])

    @pl.when(c != 0)
    def _():
        off = pl.multiple_of(NS * cpt0 + s * cpt1, GRP)
        pltpu.sync_copy(dsti.at[pl.ds(off, cpt1)], dst_v.at[pl.ds(0, cpt1)])

    plsc.subcore_barrier()

    def body(ci, carry):
        pltpu.sync_copy(ones_v, dacc.at[dst_v.at[ci]], add=True)
        return carry

    lax.fori_loop(0, nch, body, 0)
    plsc.subcore_barrier()
    pltpu.sync_copy(dacc.at[pl.ds(s * 640, 640)],
                    out.at[c, 0, pl.ds(s * 640, 640)])


def _sc_degree(cpt0, cpt1, dsti, ones_in, zer_in):
    f = pl.kernel(
        functools.partial(_deg_body, cpt0, cpt1),
        out_type=jax.ShapeDtypeStruct((NC, 1, NPD), F32),
        mesh=_sc_mesh(),
        scratch_types=[
            pltpu.VMEM((cpt0, CHUNK), jnp.int32),
            pltpu.VMEM((CHUNK,), F32),
            pltpu.VMEM_SHARED((NPD,), F32),
        ],
    )
    return f(dsti, ones_in, zer_in)


GRP = 8


def _agg_body(cpt0, cpt1, rpw, hs, srci, dsti, zer_hbm, out,
              sring, dring, rows0, rows1, acc, semg0, semg1, sems, semd):
    c = lax.axis_index("c")
    s = lax.axis_index("s")
    ngrp = jnp.where(c == 0, cpt0 // GRP, cpt1 // GRP)
    row0 = jnp.where(c == 0, s * cpt0, NS * cpt0 + s * cpt1)
    row0 = pl.multiple_of(row0, GRP)
    pltpu.sync_copy(zer_hbm, acc.at[pl.ds(s * rpw, rpw)])
    plsc.subcore_barrier()

    pltpu.sync_copy(srci.at[pl.ds(row0, GRP)], sring.at[pl.ds(0, GRP)])
    pltpu.sync_copy(dsti.at[pl.ds(row0, GRP)], dring.at[pl.ds(0, GRP)])
    pltpu.async_copy(hs.at[sring.at[0]], rows0, semg0)

    def _ring_prefetch(g):
        q = lax.rem(g + 1, 2)
        src_off = pl.multiple_of(row0 + (g + 1) * GRP, GRP)
        dst_off = pl.multiple_of(q * GRP, GRP)
        pltpu.async_copy(srci.at[pl.ds(src_off, GRP)],
                         sring.at[pl.ds(dst_off, GRP)], sems)
        pltpu.async_copy(dsti.at[pl.ds(src_off, GRP)],
                         dring.at[pl.ds(dst_off, GRP)], semd)

    def _ring_wait():
        pltpu.make_async_copy(srci.at[pl.ds(0, GRP)],
                              sring.at[pl.ds(0, GRP)], sems).wait()
        pltpu.make_async_copy(dsti.at[pl.ds(0, GRP)],
                              dring.at[pl.ds(0, GRP)], semd).wait()

    def body0(g, carry):
        p = lax.rem(g, 2)
        q = 1 - p

        @pl.when(g + 1 < ngrp)
        def _():
            _ring_prefetch(g)

        for k in range(GRP):
            buf, sem = (rows0, semg0) if k % 2 == 0 else (rows1, semg1)
            nbuf, nsem = (rows1, semg1) if k % 2 == 0 else (rows0, semg0)
            pltpu.make_async_copy(hs.at[sring.at[0]], buf, sem).wait()
            if k == GRP - 1:
                @pl.when(g + 1 < ngrp)
                def _():
                    _ring_wait()
                    pltpu.async_copy(hs.at[sring.at[q * GRP]], nbuf, nsem)
            else:
                pltpu.async_copy(hs.at[sring.at[p * GRP + k + 1]], nbuf, nsem)
            pltpu.sync_copy(buf, acc.at[dring.at[p * GRP + k]], add=True)
        return carry

    def body1(g, carry):
        p = lax.rem(g, 2)
        q = 1 - p

        @pl.when(g + 1 < ngrp)
        def _():
            _ring_prefetch(g)

        for k in range(GRP):
            pltpu.make_async_copy(hs.at[sring.at[0]], rows0, semg0).wait()
            pltpu.sync_copy(rows0, acc.at[dring.at[p * GRP + k]], add=True)
            if k == GRP - 1:
                @pl.when(g + 1 < ngrp)
                def _():
                    _ring_wait()
                    pltpu.async_copy(hs.at[sring.at[q * GRP]], rows0, semg0)
            else:
                pltpu.async_copy(hs.at[sring.at[p * GRP + k + 1]], rows0,
                                 semg0)
        return carry

    @pl.when(c == 0)
    def _():
        lax.fori_loop(0, ngrp, body0, 0)

    @pl.when(c != 0)
    def _():
        lax.fori_loop(0, ngrp, body1, 0)

    plsc.subcore_barrier()
    pltpu.sync_copy(acc.at[pl.ds(s * rpw, rpw)],
                    out.at[c, pl.ds(s * rpw, rpw)])


def _sc_aggregate(np_, cpt0, cpt1, hs, srci, dsti, zer_in):
    rpw = np_ // NS
    f = pl.kernel(
        functools.partial(_agg_body, cpt0, cpt1, rpw),
        out_type=jax.ShapeDtypeStruct((NC, np_, 128), F32),
        mesh=_sc_mesh(),
        scratch_types=[
            pltpu.VMEM((2 * GRP, CHUNK), jnp.int32),
            pltpu.VMEM((2 * GRP, CHUNK), jnp.int32),
            pltpu.VMEM((CHUNK, 128), F32),
            pltpu.VMEM((CHUNK, 128), F32),
            pltpu.VMEM_SHARED((np_, 128), F32),
            pltpu.SemaphoreType.DMA,
            pltpu.SemaphoreType.DMA,
            pltpu.SemaphoreType.DMA,
            pltpu.SemaphoreType.DMA,
        ],
    )
    return f(hs, srci, dsti, zer_in)


def _gelu(v):
    return 0.5 * v * (1.0 + lax.erf(v * (2.0 ** -0.5)))


def _dot(a, b):
    return jnp.dot(a, b, preferred_element_type=F32,
                   precision=lax.Precision.HIGHEST)


def _scale_body(x_ref, w_ref, d0_ref, d1_ref, hs_ref, dinv_ref):
    deg = d0_ref[0, 0, :] + d1_ref[0, 0, :] + 1.0
    dv = lax.rsqrt(deg)
    dinv_ref[0, 0, :] = dv
    hs_ref[...] = dv[:, None] * _dot(x_ref[...], w_ref[...])


def _tc_scale(np_, nblk, r, xp, w0big, deg0, deg1):
    f = pl.pallas_call(
        _scale_body,
        grid=(nblk,),
        in_specs=[
            pl.BlockSpec((r, 512), lambda i: (i, 0)),
            pl.BlockSpec((512, 128), lambda i: (0, 0)),
            pl.BlockSpec((1, 1, r), lambda i: (i, 0, 0)),
            pl.BlockSpec((1, 1, r), lambda i: (i, 0, 0)),
        ],
        out_specs=[
            pl.BlockSpec((r, 128), lambda i: (i, 0)),
            pl.BlockSpec((1, 1, r), lambda i: (i, 0, 0)),
        ],
        out_shape=[
            jax.ShapeDtypeStruct((np_, 128), F32),
            jax.ShapeDtypeStruct((nblk, 1, r), F32),
        ],
        compiler_params=pltpu.CompilerParams(
            dimension_semantics=("arbitrary",)),
    )
    return f(xp, w0big, deg0, deg1)


def _stats_body(g, r, parts_ref, hs_ref, dinv_ref, bt_ref, batch_ref,
                s_ref, stats_ref):
    i = pl.program_id(0)
    dv = dinv_ref[0, 0, :]
    sv = dv[:, None] * (parts_ref[0] + parts_ref[1] + hs_ref[...]) \
        + bt_ref[0, :][None, :]
    s_ref[...] = sv
    b = batch_ref[0, 0, :]
    oh = (lax.broadcasted_iota(jnp.int32, (g, r), 0) == b[None, :]) \
        .astype(F32)
    a1 = _dot(oh, sv)
    a2 = _dot(oh, sv * sv)
    cnt = jnp.sum(oh, axis=1)
    new = jnp.stack([a1, a2, jnp.broadcast_to(cnt[:, None], (g, 128))])

    @pl.when(i == 0)
    def _():
        stats_ref[...] = new

    @pl.when(i > 0)
    def _():
        stats_ref[...] += new


def _tc_stats(g, np_, nblk, r, parts, hs, dinv, bt, batch3):
    f = pl.pallas_call(
        functools.partial(_stats_body, g, r),
        grid=(nblk,),
        in_specs=[
            pl.BlockSpec((2, r, 128), lambda i: (0, i, 0)),
            pl.BlockSpec((r, 128), lambda i: (i, 0)),
            pl.BlockSpec((1, 1, r), lambda i: (i, 0, 0)),
            pl.BlockSpec((1, 128), lambda i: (0, 0)),
            pl.BlockSpec((1, 1, r), lambda i: (i, 0, 0)),
        ],
        out_specs=[
            pl.BlockSpec((r, 128), lambda i: (i, 0)),
            pl.BlockSpec((3, g, 128), lambda i: (0, 0, 0)),
        ],
        out_shape=[
            jax.ShapeDtypeStruct((np_, 128), F32),
            jax.ShapeDtypeStruct((3, g, 128), F32),
        ],
        compiler_params=pltpu.CompilerParams(
            dimension_semantics=("arbitrary",)),
    )
    return f(parts, hs, dinv, bt, batch3)


def _norm_scale_off(stats, gw, gb, gms):
    cnt = jnp.maximum(stats[2], 1.0)
    mean = stats[0] / cnt
    e2 = stats[1] / cnt
    ms = gms[0, :][None, :]
    var = e2 - mean * mean * ms * (2.0 - ms)
    rs = lax.rsqrt(var + EPS)
    w = gw[0, :][None, :]
    scale = w * rs
    off = gb[0, :][None, :] - w * ms * mean * rs
    return scale, off


def _row_gather(batch_ref, g, r, scale, off):
    b = batch_ref[0, 0, :]
    oht = (b[:, None] == lax.broadcasted_iota(jnp.int32, (r, g), 1)) \
        .astype(F32)
    return _dot(oht, scale), _dot(oht, off)


def _norm0_body(g, r, s_ref, stats_ref, batch_ref, dinv_ref,
                gw_ref, gb_ref, gms_ref, w1_ref, res_ref, hs1_ref):
    scale, off = _norm_scale_off(stats_ref[...], gw_ref, gb_ref, gms_ref)
    sc_r, off_r = _row_gather(batch_ref, g, r, scale, off)
    gv = _gelu(s_ref[...] * sc_r + off_r)
    res_ref[...] = gv
    hs1_ref[...] = dinv_ref[0, 0, :][:, None] * _dot(gv, w1_ref[...])


def _tc_norm0(g, np_, nblk, r, s, stats, batch3, dinv, gw, gb, gms, w1big):
    f = pl.pallas_call(
        functools.partial(_norm0_body, g, r),
        grid=(nblk,),
        in_specs=[
            pl.BlockSpec((r, 128), lambda i: (i, 0)),
            pl.BlockSpec((3, g, 128), lambda i: (0, 0, 0)),
            pl.BlockSpec((1, 1, r), lambda i: (i, 0, 0)),
            pl.BlockSpec((1, 1, r), lambda i: (i, 0, 0)),
            pl.BlockSpec((1, 128), lambda i: (0, 0)),
            pl.BlockSpec((1, 128), lambda i: (0, 0)),
            pl.BlockSpec((1, 128), lambda i: (0, 0)),
            pl.BlockSpec((128, 128), lambda i: (0, 0)),
        ],
        out_specs=[
            pl.BlockSpec((r, 128), lambda i: (i, 0)),
            pl.BlockSpec((r, 128), lambda i: (i, 0)),
        ],
        out_shape=[
            jax.ShapeDtypeStruct((np_, 128), F32),
            jax.ShapeDtypeStruct((np_, 128), F32),
        ],
        compiler_params=pltpu.CompilerParams(
            dimension_semantics=("arbitrary",)),
    )
    return f(s, stats, batch3, dinv, gw, gb, gms, w1big)


def _final_body(g, r, s_ref, stats_ref, batch_ref, res_ref,
                gw_ref, gb_ref, gms_ref, hw0_ref, hb0_ref, hw1_ref, hb1_ref,
                z_ref):
    scale, off = _norm_scale_off(stats_ref[...], gw_ref, gb_ref, gms_ref)
    sc_r, off_r = _row_gather(batch_ref, g, r, scale, off)
    h1 = _gelu(s_ref[...] * sc_r + off_r) + res_ref[...]
    t1 = _gelu(_dot(h1, hw0_ref[...]) + hb0_ref[0, :][None, :])
    z_ref[...] = _dot(t1, hw1_ref[...]) + hb1_ref[0, :][None, :]


def _tc_final(g, np_, nblk, r, s, stats, batch3, res0,
              gw, gb, gms, hw0big, hb0t, hw1big, hb1t):
    f = pl.pallas_call(
        functools.partial(_final_body, g, r),
        grid=(nblk,),
        in_specs=[
            pl.BlockSpec((r, 128), lambda i: (i, 0)),
            pl.BlockSpec((3, g, 128), lambda i: (0, 0, 0)),
            pl.BlockSpec((1, 1, r), lambda i: (i, 0, 0)),
            pl.BlockSpec((r, 128), lambda i: (i, 0)),
            pl.BlockSpec((1, 128), lambda i: (0, 0)),
            pl.BlockSpec((1, 128), lambda i: (0, 0)),
            pl.BlockSpec((1, 128), lambda i: (0, 0)),
            pl.BlockSpec((128, 128), lambda i: (0, 0)),
            pl.BlockSpec((1, 128), lambda i: (0, 0)),
            pl.BlockSpec((128, 128), lambda i: (0, 0)),
            pl.BlockSpec((1, 128), lambda i: (0, 0)),
        ],
        out_specs=[pl.BlockSpec((r, 128), lambda i: (i, 0))],
        out_shape=[jax.ShapeDtypeStruct((np_, 128), F32)],
        compiler_params=pltpu.CompilerParams(
            dimension_semantics=("arbitrary",)),
    )
    return f(s, stats, batch3, res0, gw, gb, gms, hw0big, hb0t, hw1big, hb1t)[0]


def kernel(x, batch, edge_index, W0, b0, gn0_w, gn0_b, gn0_ms,
           W1, b1, gn1_w, gn1_b, gn1_ms, hW0, hb0, hW1, hb1):
    n, t, d = x.shape
    h = W0.shape[1]
    e = edge_index.shape[1]
    g = 16
    out_f = hW1.shape[1]

    r = 1280
    np_ = ((n + 1 + r - 1) // r) * r
    nblk = np_ // r
    ncht = (e + NS * CHUNK - 1) // (NS * CHUNK)
    cpt0 = max(GRP, (ncht * 5) // 8 // GRP * GRP)
    cpt1 = max(GRP, (ncht - cpt0 + GRP - 1) // GRP * GRP)
    e_pad = NS * (cpt0 + cpt1) * CHUNK

    x2 = x.reshape(n, t * d)
    xp = jnp.zeros((np_, t * d), F32).at[:n].set(x2)
    batchp = jnp.full((np_,), g, jnp.int32).at[:n].set(batch.astype(jnp.int32))
    batch3 = batchp.reshape(nblk, 1, r)
    ei = edge_index.astype(jnp.int32)
    pad_idx = n + jnp.arange(e_pad - e, dtype=jnp.int32) % (np_ - n)
    srcp = jnp.concatenate([ei[0], pad_idx])
    dstp = jnp.concatenate([ei[1], pad_idx])

    srci = srcp.reshape(NS * (cpt0 + cpt1), CHUNK)
    dsti = dstp.reshape(NS * (cpt0 + cpt1), CHUNK)

    eye_t = jnp.eye(t, dtype=F32)
    w0big = jnp.kron(eye_t, W0)
    w1big = jnp.kron(eye_t, W1)
    hw0big = jnp.kron(eye_t, hW0)
    hw1big = jnp.zeros((t * h, 128), F32).at[:, :t * out_f].set(
        jnp.kron(eye_t, hW1))
    b0t = jnp.tile(b0, t).reshape(1, t * h)
    b1t = jnp.tile(b1, t).reshape(1, t * h)
    gw0 = jnp.tile(gn0_w, t).reshape(1, t * h)
    gb0 = jnp.tile(gn0_b, t).reshape(1, t * h)
    gm0 = jnp.tile(gn0_ms, t).reshape(1, t * h)
    gw1 = jnp.tile(gn1_w, t).reshape(1, t * h)
    gb1 = jnp.tile(gn1_b, t).reshape(1, t * h)
    gm1 = jnp.tile(gn1_ms, t).reshape(1, t * h)
    hb0t = jnp.tile(hb0, t).reshape(1, t * h)
    hb1t = jnp.zeros((1, 128), F32).at[0, :t * out_f].set(jnp.tile(hb1, t))

    rpw = np_ // NS
    zer128 = jnp.zeros((rpw, 128), F32)
    zer1 = jnp.zeros((640,), F32)
    ones1 = jnp.ones((CHUNK,), F32)

    degp = _sc_degree(cpt0, cpt1, dsti, ones1, zer1)
    deg0 = degp[0, 0, :np_].reshape(nblk, 1, r)
    deg1 = degp[1, 0, :np_].reshape(nblk, 1, r)

    hs0, dinv = _tc_scale(np_, nblk, r, xp, w0big, deg0, deg1)
    parts0 = _sc_aggregate(np_, cpt0, cpt1, hs0, srci, dsti, zer128)
    s0, stats0 = _tc_stats(g, np_, nblk, r, parts0, hs0, dinv, b0t, batch3)
    res0, hs1 = _tc_norm0(g, np_, nblk, r, s0, stats0, batch3, dinv,
                          gw0, gb0, gm0, w1big)
    parts1 = _sc_aggregate(np_, cpt0, cpt1, hs1, srci, dsti, zer128)
    s1, stats1 = _tc_stats(g, np_, nblk, r, parts1, hs1, dinv, b1t, batch3)
    z = _tc_final(g, np_, nblk, r, s1, stats1, batch3, res0,
                  gw1, gb1, gm1, hw0big, hb0t, hw1big, hb1t)

    return z[:n, :t * out_f].reshape(n, t, out_f)


# speedup vs baseline: 2.2526x; 1.0473x over previous
import functools

import jax
import jax.numpy as jnp
from jax import lax
from jax.experimental import pallas as pl
from jax.experimental.pallas import tpu as pltpu
from jax.experimental.pallas import tpu_sc as plsc

NC = 2
NS = 16
NW = NC * NS
CHUNK = 128
EPS = 1e-5

F32 = jnp.float32


def _sc_mesh():
    return plsc.VectorSubcoreMesh(
        core_axis_name="c", subcore_axis_name="s",
        num_cores=NC, num_subcores=NS)


NPD = NS * 640


def _deg_body(cpt0, cpt1, dsti, ones_hbm, zer_hbm, out, dst_v, ones_v, dacc):
    c = lax.axis_index("c")
    s = lax.axis_index("s")
    nch = jnp.where(c == 0, cpt0, cpt1)
    pltpu.sync_copy(zer_hbm, dacc.at[pl.ds(s * 640, 640)])
    pltpu.sync_copy(ones_hbm, ones_v)

    @pl.when(c == 0)
    def _():
        off = pl.multiple_of(s * cpt0, GRP)
        pltpu.sync_copy(dsti.at[pl.ds(off, cpt0)], dst_v.at[pl.ds(0, cpt0)])

    @pl.when(c != 0)
    def _():
        off = pl.multiple_of(NS * cpt0 + s * cpt1, GRP)
        pltpu.sync_copy(dsti.at[pl.ds(off, cpt1)], dst_v.at[pl.ds(0, cpt1)])

    plsc.subcore_barrier()

    def body(ci, carry):
        pltpu.sync_copy(ones_v, dacc.at[dst_v.at[ci]], add=True)
        return carry

    lax.fori_loop(0, nch, body, 0)
    plsc.subcore_barrier()
    pltpu.sync_copy(dacc.at[pl.ds(s * 640, 640)],
                    out.at[c, 0, pl.ds(s * 640, 640)])


def _sc_degree(cpt0, cpt1, dsti, ones_in, zer_in):
    f = pl.kernel(
        functools.partial(_deg_body, cpt0, cpt1),
        out_type=jax.ShapeDtypeStruct((NC, 1, NPD), F32),
        mesh=_sc_mesh(),
        scratch_types=[
            pltpu.VMEM((cpt0, CHUNK), jnp.int32),
            pltpu.VMEM((CHUNK,), F32),
            pltpu.VMEM_SHARED((NPD,), F32),
        ],
    )
    return f(dsti, ones_in, zer_in)


GRP = 8


def _agg_body(cpt0, cpt1, rpw, hs, srci, dsti, zer_hbm, out,
              sring, dring, rows0, rows1, acc, semg0, semg1, sems, semd):
    c = lax.axis_index("c")
    s = lax.axis_index("s")
    ngrp = jnp.where(c == 0, cpt0 // GRP, cpt1 // GRP)
    row0 = jnp.where(c == 0, s * cpt0, NS * cpt0 + s * cpt1)
    row0 = pl.multiple_of(row0, GRP)
    pltpu.sync_copy(zer_hbm, acc.at[pl.ds(s * rpw, rpw)])
    plsc.subcore_barrier()

    pltpu.sync_copy(srci.at[pl.ds(row0, GRP)], sring.at[pl.ds(0, GRP)])
    pltpu.sync_copy(dsti.at[pl.ds(row0, GRP)], dring.at[pl.ds(0, GRP)])
    pltpu.async_copy(hs.at[sring.at[0]], rows0, semg0)

    def _ring_prefetch(g):
        q = lax.rem(g + 1, 2)
        src_off = pl.multiple_of(row0 + (g + 1) * GRP, GRP)
        dst_off = pl.multiple_of(q * GRP, GRP)
        pltpu.async_copy(srci.at[pl.ds(src_off, GRP)],
                         sring.at[pl.ds(dst_off, GRP)], sems)
        pltpu.async_copy(dsti.at[pl.ds(src_off, GRP)],
                         dring.at[pl.ds(dst_off, GRP)], semd)

    def _ring_wait():
        pltpu.make_async_copy(srci.at[pl.ds(0, GRP)],
                              sring.at[pl.ds(0, GRP)], sems).wait()
        pltpu.make_async_copy(dsti.at[pl.ds(0, GRP)],
                              dring.at[pl.ds(0, GRP)], semd).wait()

    def body0(g, carry):
        p = lax.rem(g, 2)
        q = 1 - p

        @pl.when(g + 1 < ngrp)
        def _():
            _ring_prefetch(g)

        for k in range(GRP):
            buf, sem = (rows0, semg0) if k % 2 == 0 else (rows1, semg1)
            nbuf, nsem = (rows1, semg1) if k % 2 == 0 else (rows0, semg0)
            pltpu.make_async_copy(hs.at[sring.at[0]], buf, sem).wait()
            if k == GRP - 1:
                @pl.when(g + 1 < ngrp)
                def _():
                    _ring_wait()
                    pltpu.async_copy(hs.at[sring.at[q * GRP]], nbuf, nsem)
            else:
                pltpu.async_copy(hs.at[sring.at[p * GRP + k + 1]], nbuf, nsem)
            pltpu.sync_copy(buf, acc.at[dring.at[p * GRP + k]], add=True)
        return carry

    def body1(g, carry):
        p = lax.rem(g, 2)
        q = 1 - p

        @pl.when(g + 1 < ngrp)
        def _():
            _ring_prefetch(g)

        for k in range(GRP):
            pltpu.make_async_copy(hs.at[sring.at[0]], rows0, semg0).wait()
            pltpu.sync_copy(rows0, acc.at[dring.at[p * GRP + k]], add=True)
            if k == GRP - 1:
                @pl.when(g + 1 < ngrp)
                def _():
                    _ring_wait()
                    pltpu.async_copy(hs.at[sring.at[q * GRP]], rows0, semg0)
            else:
                pltpu.async_copy(hs.at[sring.at[p * GRP + k + 1]], rows0,
                                 semg0)
        return carry

    @pl.when(c == 0)
    def _():
        lax.fori_loop(0, ngrp, body0, 0)

    @pl.when(c != 0)
    def _():
        lax.fori_loop(0, ngrp, body0, 0)

    plsc.subcore_barrier()
    pltpu.sync_copy(acc.at[pl.ds(s * rpw, rpw)],
                    out.at[c, pl.ds(s * rpw, rpw)])


def _sc_aggregate(np_, cpt0, cpt1, hs, srci, dsti, zer_in):
    rpw = np_ // NS
    f = pl.kernel(
        functools.partial(_agg_body, cpt0, cpt1, rpw),
        out_type=jax.ShapeDtypeStruct((NC, np_, 128), F32),
        mesh=_sc_mesh(),
        scratch_types=[
            pltpu.VMEM((2 * GRP, CHUNK), jnp.int32),
            pltpu.VMEM((2 * GRP, CHUNK), jnp.int32),
            pltpu.VMEM((CHUNK, 128), F32),
            pltpu.VMEM((CHUNK, 128), F32),
            pltpu.VMEM_SHARED((np_, 128), F32),
            pltpu.SemaphoreType.DMA,
            pltpu.SemaphoreType.DMA,
            pltpu.SemaphoreType.DMA,
            pltpu.SemaphoreType.DMA,
        ],
    )
    return f(hs, srci, dsti, zer_in)


def _gelu(v):
    return 0.5 * v * (1.0 + lax.erf(v * (2.0 ** -0.5)))


def _dot(a, b):
    return jnp.dot(a, b, preferred_element_type=F32,
                   precision=lax.Precision.HIGHEST)


def _scale_body(x_ref, w_ref, d0_ref, d1_ref, hs_ref, dinv_ref):
    deg = d0_ref[0, 0, :] + d1_ref[0, 0, :] + 1.0
    dv = lax.rsqrt(deg)
    dinv_ref[0, 0, :] = dv
    hs_ref[...] = dv[:, None] * _dot(x_ref[...], w_ref[...])


def _tc_scale(np_, nblk, r, xp, w0big, deg0, deg1):
    f = pl.pallas_call(
        _scale_body,
        grid=(nblk,),
        in_specs=[
            pl.BlockSpec((r, 512), lambda i: (i, 0)),
            pl.BlockSpec((512, 128), lambda i: (0, 0)),
            pl.BlockSpec((1, 1, r), lambda i: (i, 0, 0)),
            pl.BlockSpec((1, 1, r), lambda i: (i, 0, 0)),
        ],
        out_specs=[
            pl.BlockSpec((r, 128), lambda i: (i, 0)),
            pl.BlockSpec((1, 1, r), lambda i: (i, 0, 0)),
        ],
        out_shape=[
            jax.ShapeDtypeStruct((np_, 128), F32),
            jax.ShapeDtypeStruct((nblk, 1, r), F32),
        ],
        compiler_params=pltpu.CompilerParams(
            dimension_semantics=("arbitrary",)),
    )
    return f(xp, w0big, deg0, deg1)


def _stats_body(g, r, parts_ref, hs_ref, dinv_ref, bt_ref, batch_ref,
                s_ref, stats_ref):
    i = pl.program_id(0)
    dv = dinv_ref[0, 0, :]
    sv = dv[:, None] * (parts_ref[0] + parts_ref[1] + hs_ref[...]) \
        + bt_ref[0, :][None, :]
    s_ref[...] = sv
    b = batch_ref[0, 0, :]
    oh = (lax.broadcasted_iota(jnp.int32, (g, r), 0) == b[None, :]) \
        .astype(F32)
    a1 = _dot(oh, sv)
    a2 = _dot(oh, sv * sv)
    cnt = jnp.sum(oh, axis=1)
    new = jnp.stack([a1, a2, jnp.broadcast_to(cnt[:, None], (g, 128))])

    @pl.when(i == 0)
    def _():
        stats_ref[...] = new

    @pl.when(i > 0)
    def _():
        stats_ref[...] += new


def _tc_stats(g, np_, nblk, r, parts, hs, dinv, bt, batch3):
    f = pl.pallas_call(
        functools.partial(_stats_body, g, r),
        grid=(nblk,),
        in_specs=[
            pl.BlockSpec((2, r, 128), lambda i: (0, i, 0)),
            pl.BlockSpec((r, 128), lambda i: (i, 0)),
            pl.BlockSpec((1, 1, r), lambda i: (i, 0, 0)),
            pl.BlockSpec((1, 128), lambda i: (0, 0)),
            pl.BlockSpec((1, 1, r), lambda i: (i, 0, 0)),
        ],
        out_specs=[
            pl.BlockSpec((r, 128), lambda i: (i, 0)),
            pl.BlockSpec((3, g, 128), lambda i: (0, 0, 0)),
        ],
        out_shape=[
            jax.ShapeDtypeStruct((np_, 128), F32),
            jax.ShapeDtypeStruct((3, g, 128), F32),
        ],
        compiler_params=pltpu.CompilerParams(
            dimension_semantics=("arbitrary",)),
    )
    return f(parts, hs, dinv, bt, batch3)


def _norm_scale_off(stats, gw, gb, gms):
    cnt = jnp.maximum(stats[2], 1.0)
    mean = stats[0] / cnt
    e2 = stats[1] / cnt
    ms = gms[0, :][None, :]
    var = e2 - mean * mean * ms * (2.0 - ms)
    rs = lax.rsqrt(var + EPS)
    w = gw[0, :][None, :]
    scale = w * rs
    off = gb[0, :][None, :] - w * ms * mean * rs
    return scale, off


def _row_gather(batch_ref, g, r, scale, off):
    b = batch_ref[0, 0, :]
    oht = (b[:, None] == lax.broadcasted_iota(jnp.int32, (r, g), 1)) \
        .astype(F32)
    return _dot(oht, scale), _dot(oht, off)


def _norm0_body(g, r, s_ref, stats_ref, batch_ref, dinv_ref,
                gw_ref, gb_ref, gms_ref, w1_ref, res_ref, hs1_ref):
    scale, off = _norm_scale_off(stats_ref[...], gw_ref, gb_ref, gms_ref)
    sc_r, off_r = _row_gather(batch_ref, g, r, scale, off)
    gv = _gelu(s_ref[...] * sc_r + off_r)
    res_ref[...] = gv
    hs1_ref[...] = dinv_ref[0, 0, :][:, None] * _dot(gv, w1_ref[...])


def _tc_norm0(g, np_, nblk, r, s, stats, batch3, dinv, gw, gb, gms, w1big):
    f = pl.pallas_call(
        functools.partial(_norm0_body, g, r),
        grid=(nblk,),
        in_specs=[
            pl.BlockSpec((r, 128), lambda i: (i, 0)),
            pl.BlockSpec((3, g, 128), lambda i: (0, 0, 0)),
            pl.BlockSpec((1, 1, r), lambda i: (i, 0, 0)),
            pl.BlockSpec((1, 1, r), lambda i: (i, 0, 0)),
            pl.BlockSpec((1, 128), lambda i: (0, 0)),
            pl.BlockSpec((1, 128), lambda i: (0, 0)),
            pl.BlockSpec((1, 128), lambda i: (0, 0)),
            pl.BlockSpec((128, 128), lambda i: (0, 0)),
        ],
        out_specs=[
            pl.BlockSpec((r, 128), lambda i: (i, 0)),
            pl.BlockSpec((r, 128), lambda i: (i, 0)),
        ],
        out_shape=[
            jax.ShapeDtypeStruct((np_, 128), F32),
            jax.ShapeDtypeStruct((np_, 128), F32),
        ],
        compiler_params=pltpu.CompilerParams(
            dimension_semantics=("arbitrary",)),
    )
    return f(s, stats, batch3, dinv, gw, gb, gms, w1big)


def _final_body(g, r, s_ref, stats_ref, batch_ref, res_ref,
                gw_ref, gb_ref, gms_ref, hw0_ref, hb0_ref, hw1_ref, hb1_ref,
                z_ref):
    scale, off = _norm_scale_off(stats_ref[...], gw_ref, gb_ref, gms_ref)
    sc_r, off_r = _row_gather(batch_ref, g, r, scale, off)
    h1 = _gelu(s_ref[...] * sc_r + off_r) + res_ref[...]
    t1 = _gelu(_dot(h1, hw0_ref[...]) + hb0_ref[0, :][None, :])
    z_ref[...] = _dot(t1, hw1_ref[...]) + hb1_ref[0, :][None, :]


def _tc_final(g, np_, nblk, r, s, stats, batch3, res0,
              gw, gb, gms, hw0big, hb0t, hw1big, hb1t):
    f = pl.pallas_call(
        functools.partial(_final_body, g, r),
        grid=(nblk,),
        in_specs=[
            pl.BlockSpec((r, 128), lambda i: (i, 0)),
            pl.BlockSpec((3, g, 128), lambda i: (0, 0, 0)),
            pl.BlockSpec((1, 1, r), lambda i: (i, 0, 0)),
            pl.BlockSpec((r, 128), lambda i: (i, 0)),
            pl.BlockSpec((1, 128), lambda i: (0, 0)),
            pl.BlockSpec((1, 128), lambda i: (0, 0)),
            pl.BlockSpec((1, 128), lambda i: (0, 0)),
            pl.BlockSpec((128, 128), lambda i: (0, 0)),
            pl.BlockSpec((1, 128), lambda i: (0, 0)),
            pl.BlockSpec((128, 128), lambda i: (0, 0)),
            pl.BlockSpec((1, 128), lambda i: (0, 0)),
        ],
        out_specs=[pl.BlockSpec((r, 128), lambda i: (i, 0))],
        out_shape=[jax.ShapeDtypeStruct((np_, 128), F32)],
        compiler_params=pltpu.CompilerParams(
            dimension_semantics=("arbitrary",)),
    )
    return f(s, stats, batch3, res0, gw, gb, gms, hw0big, hb0t, hw1big, hb1t)[0]


def kernel(x, batch, edge_index, W0, b0, gn0_w, gn0_b, gn0_ms,
           W1, b1, gn1_w, gn1_b, gn1_ms, hW0, hb0, hW1, hb1):
    n, t, d = x.shape
    h = W0.shape[1]
    e = edge_index.shape[1]
    g = 16
    out_f = hW1.shape[1]

    r = 1280
    np_ = ((n + 1 + r - 1) // r) * r
    nblk = np_ // r
    ncht = (e + NS * CHUNK - 1) // (NS * CHUNK)
    cpt0 = max(GRP, (ncht * 9) // 16 // GRP * GRP)
    cpt1 = max(GRP, (ncht - cpt0 + GRP - 1) // GRP * GRP)
    e_pad = NS * (cpt0 + cpt1) * CHUNK

    x2 = x.reshape(n, t * d)
    xp = jnp.zeros((np_, t * d), F32).at[:n].set(x2)
    batchp = jnp.full((np_,), g, jnp.int32).at[:n].set(batch.astype(jnp.int32))
    batch3 = batchp.reshape(nblk, 1, r)
    ei = edge_index.astype(jnp.int32)
    pad_idx = n + jnp.arange(e_pad - e, dtype=jnp.int32) % (np_ - n)
    srcp = jnp.concatenate([ei[0], pad_idx])
    dstp = jnp.concatenate([ei[1], pad_idx])

    srci = srcp.reshape(NS * (cpt0 + cpt1), CHUNK)
    dsti = dstp.reshape(NS * (cpt0 + cpt1), CHUNK)

    eye_t = jnp.eye(t, dtype=F32)
    w0big = jnp.kron(eye_t, W0)
    w1big = jnp.kron(eye_t, W1)
    hw0big = jnp.kron(eye_t, hW0)
    hw1big = jnp.zeros((t * h, 128), F32).at[:, :t * out_f].set(
        jnp.kron(eye_t, hW1))
    b0t = jnp.tile(b0, t).reshape(1, t * h)
    b1t = jnp.tile(b1, t).reshape(1, t * h)
    gw0 = jnp.tile(gn0_w, t).reshape(1, t * h)
    gb0 = jnp.tile(gn0_b, t).reshape(1, t * h)
    gm0 = jnp.tile(gn0_ms, t).reshape(1, t * h)
    gw1 = jnp.tile(gn1_w, t).reshape(1, t * h)
    gb1 = jnp.tile(gn1_b, t).reshape(1, t * h)
    gm1 = jnp.tile(gn1_ms, t).reshape(1, t * h)
    hb0t = jnp.tile(hb0, t).reshape(1, t * h)
    hb1t = jnp.zeros((1, 128), F32).at[0, :t * out_f].set(jnp.tile(hb1, t))

    rpw = np_ // NS
    zer128 = jnp.zeros((rpw, 128), F32)
    zer1 = jnp.zeros((640,), F32)
    ones1 = jnp.ones((CHUNK,), F32)

    degp = _sc_degree(cpt0, cpt1, dsti, ones1, zer1)
    deg0 = degp[0, 0, :np_].reshape(nblk, 1, r)
    deg1 = degp[1, 0, :np_].reshape(nblk, 1, r)

    hs0, dinv = _tc_scale(np_, nblk, r, xp, w0big, deg0, deg1)
    parts0 = _sc_aggregate(np_, cpt0, cpt1, hs0, srci, dsti, zer128)
    s0, stats0 = _tc_stats(g, np_, nblk, r, parts0, hs0, dinv, b0t, batch3)
    res0, hs1 = _tc_norm0(g, np_, nblk, r, s0, stats0, batch3, dinv,
                          gw0, gb0, gm0, w1big)
    parts1 = _sc_aggregate(np_, cpt0, cpt1, hs1, srci, dsti, zer128)
    s1, stats1 = _tc_stats(g, np_, nblk, r, parts1, hs1, dinv, b1t, batch3)
    z = _tc_final(g, np_, nblk, r, s1, stats1, batch3, res0,
                  gw1, gb1, gm1, hw0big, hb0t, hw1big, hb1t)

    return z[:n, :t * out_f].reshape(n, t, out_f)


# speedup vs baseline: 2.2606x; 1.0035x over previous
import functools

import jax
import jax.numpy as jnp
from jax import lax
from jax.experimental import pallas as pl
from jax.experimental.pallas import tpu as pltpu
from jax.experimental.pallas import tpu_sc as plsc

NC = 2
NS = 16
NW = NC * NS
CHUNK = 128
EPS = 1e-5

F32 = jnp.float32


def _sc_mesh():
    return plsc.VectorSubcoreMesh(
        core_axis_name="c", subcore_axis_name="s",
        num_cores=NC, num_subcores=NS)


NPD = NS * 640


def _deg_body(cpt0, cpt1, dsti, ones_hbm, zer_hbm, out, dst_v, ones_v, dacc):
    c = lax.axis_index("c")
    s = lax.axis_index("s")
    nch = jnp.where(c == 0, cpt0, cpt1)
    pltpu.sync_copy(zer_hbm, dacc.at[pl.ds(s * 640, 640)])
    pltpu.sync_copy(ones_hbm, ones_v)

    @pl.when(c == 0)
    def _():
        off = pl.multiple_of(s * cpt0, GRP)
        pltpu.sync_copy(dsti.at[pl.ds(off, cpt0)], dst_v.at[pl.ds(0, cpt0)])

    @pl.when(c != 0)
    def _():
        off = pl.multiple_of(NS * cpt0 + s * cpt1, GRP)
        pltpu.sync_copy(dsti.at[pl.ds(off, cpt1)], dst_v.at[pl.ds(0, cpt1)])

    plsc.subcore_barrier()

    def body(ci, carry):
        pltpu.sync_copy(ones_v, dacc.at[dst_v.at[ci]], add=True)
        return carry

    lax.fori_loop(0, nch, body, 0)
    plsc.subcore_barrier()
    pltpu.sync_copy(dacc.at[pl.ds(s * 640, 640)],
                    out.at[c, 0, pl.ds(s * 640, 640)])


def _sc_degree(cpt0, cpt1, dsti, ones_in, zer_in):
    f = pl.kernel(
        functools.partial(_deg_body, cpt0, cpt1),
        out_type=jax.ShapeDtypeStruct((NC, 1, NPD), F32),
        mesh=_sc_mesh(),
        scratch_types=[
            pltpu.VMEM((cpt0, CHUNK), jnp.int32),
            pltpu.VMEM((CHUNK,), F32),
            pltpu.VMEM_SHARED((NPD,), F32),
        ],
    )
    return f(dsti, ones_in, zer_in)


GRP = 8


def _agg_body(cpt0, cpt1, rpw, hs, srci, dsti, zer_hbm, out,
              sring, dring, rows0, rows1, acc, semg0, semg1, sems, semd):
    c = lax.axis_index("c")
    s = lax.axis_index("s")
    ngrp = jnp.where(c == 0, cpt0 // GRP, cpt1 // GRP)
    row0 = jnp.where(c == 0, s * cpt0, NS * cpt0 + s * cpt1)
    row0 = pl.multiple_of(row0, GRP)
    pltpu.sync_copy(zer_hbm, acc.at[pl.ds(s * rpw, rpw)])
    plsc.subcore_barrier()

    pltpu.sync_copy(srci.at[pl.ds(row0, GRP)], sring.at[pl.ds(0, GRP)])
    pltpu.sync_copy(dsti.at[pl.ds(row0, GRP)], dring.at[pl.ds(0, GRP)])
    pltpu.async_copy(hs.at[sring.at[0]], rows0, semg0)

    def _ring_prefetch(g):
        q = lax.rem(g + 1, 2)
        src_off = pl.multiple_of(row0 + (g + 1) * GRP, GRP)
        dst_off = pl.multiple_of(q * GRP, GRP)
        pltpu.async_copy(srci.at[pl.ds(src_off, GRP)],
                         sring.at[pl.ds(dst_off, GRP)], sems)
        pltpu.async_copy(dsti.at[pl.ds(src_off, GRP)],
                         dring.at[pl.ds(dst_off, GRP)], semd)

    def _ring_wait():
        pltpu.make_async_copy(srci.at[pl.ds(0, GRP)],
                              sring.at[pl.ds(0, GRP)], sems).wait()
        pltpu.make_async_copy(dsti.at[pl.ds(0, GRP)],
                              dring.at[pl.ds(0, GRP)], semd).wait()

    def body0(g, carry):
        p = lax.rem(g, 2)
        q = 1 - p

        @pl.when(g + 1 < ngrp)
        def _():
            _ring_prefetch(g)

        for k in range(GRP):
            buf, sem = (rows0, semg0) if k % 2 == 0 else (rows1, semg1)
            nbuf, nsem = (rows1, semg1) if k % 2 == 0 else (rows0, semg0)
            pltpu.make_async_copy(hs.at[sring.at[0]], buf, sem).wait()
            if k == GRP - 1:
                @pl.when(g + 1 < ngrp)
                def _():
                    _ring_wait()
                    pltpu.async_copy(hs.at[sring.at[q * GRP]], nbuf, nsem)
            else:
                pltpu.async_copy(hs.at[sring.at[p * GRP + k + 1]], nbuf, nsem)
            pltpu.sync_copy(buf, acc.at[dring.at[p * GRP + k]], add=True)
        return carry

    lax.fori_loop(0, ngrp, body0, 0)

    plsc.subcore_barrier()
    pltpu.sync_copy(acc.at[pl.ds(s * rpw, rpw)],
                    out.at[c, pl.ds(s * rpw, rpw)])


def _sc_aggregate(np_, cpt0, cpt1, hs, srci, dsti, zer_in):
    rpw = np_ // NS
    f = pl.kernel(
        functools.partial(_agg_body, cpt0, cpt1, rpw),
        out_type=jax.ShapeDtypeStruct((NC, np_, 128), F32),
        mesh=_sc_mesh(),
        scratch_types=[
            pltpu.VMEM((2 * GRP, CHUNK), jnp.int32),
            pltpu.VMEM((2 * GRP, CHUNK), jnp.int32),
            pltpu.VMEM((CHUNK, 128), F32),
            pltpu.VMEM((CHUNK, 128), F32),
            pltpu.VMEM_SHARED((np_, 128), F32),
            pltpu.SemaphoreType.DMA,
            pltpu.SemaphoreType.DMA,
            pltpu.SemaphoreType.DMA,
            pltpu.SemaphoreType.DMA,
        ],
    )
    return f(hs, srci, dsti, zer_in)


def _gelu(v):
    return 0.5 * v * (1.0 + lax.erf(v * (2.0 ** -0.5)))


def _dot(a, b):
    return jnp.dot(a, b, preferred_element_type=F32,
                   precision=lax.Precision.HIGHEST)


def _scale_body(x_ref, w_ref, d0_ref, d1_ref, hs_ref, dinv_ref):
    deg = d0_ref[0, 0, :] + d1_ref[0, 0, :] + 1.0
    dv = lax.rsqrt(deg)
    dinv_ref[0, 0, :] = dv
    hs_ref[...] = dv[:, None] * _dot(x_ref[...], w_ref[...])


def _tc_scale(np_, nblk, r, xp, w0big, deg0, deg1):
    f = pl.pallas_call(
        _scale_body,
        grid=(nblk,),
        in_specs=[
            pl.BlockSpec((r, 512), lambda i: (i, 0)),
            pl.BlockSpec((512, 128), lambda i: (0, 0)),
            pl.BlockSpec((1, 1, r), lambda i: (i, 0, 0)),
            pl.BlockSpec((1, 1, r), lambda i: (i, 0, 0)),
        ],
        out_specs=[
            pl.BlockSpec((r, 128), lambda i: (i, 0)),
            pl.BlockSpec((1, 1, r), lambda i: (i, 0, 0)),
        ],
        out_shape=[
            jax.ShapeDtypeStruct((np_, 128), F32),
            jax.ShapeDtypeStruct((nblk, 1, r), F32),
        ],
        compiler_params=pltpu.CompilerParams(
            dimension_semantics=("arbitrary",)),
    )
    return f(xp, w0big, deg0, deg1)


def _stats_body(g, r, parts_ref, hs_ref, dinv_ref, bt_ref, batch_ref,
                s_ref, stats_ref):
    i = pl.program_id(0)
    dv = dinv_ref[0, 0, :]
    sv = dv[:, None] * (parts_ref[0] + parts_ref[1] + hs_ref[...]) \
        + bt_ref[0, :][None, :]
    s_ref[...] = sv
    b = batch_ref[0, 0, :]
    oh = (lax.broadcasted_iota(jnp.int32, (g, r), 0) == b[None, :]) \
        .astype(F32)
    a1 = _dot(oh, sv)
    a2 = _dot(oh, sv * sv)
    cnt = jnp.sum(oh, axis=1)
    new = jnp.stack([a1, a2, jnp.broadcast_to(cnt[:, None], (g, 128))])

    @pl.when(i == 0)
    def _():
        stats_ref[...] = new

    @pl.when(i > 0)
    def _():
        stats_ref[...] += new


def _tc_stats(g, np_, nblk, r, parts, hs, dinv, bt, batch3):
    f = pl.pallas_call(
        functools.partial(_stats_body, g, r),
        grid=(nblk,),
        in_specs=[
            pl.BlockSpec((2, r, 128), lambda i: (0, i, 0)),
            pl.BlockSpec((r, 128), lambda i: (i, 0)),
            pl.BlockSpec((1, 1, r), lambda i: (i, 0, 0)),
            pl.BlockSpec((1, 128), lambda i: (0, 0)),
            pl.BlockSpec((1, 1, r), lambda i: (i, 0, 0)),
        ],
        out_specs=[
            pl.BlockSpec((r, 128), lambda i: (i, 0)),
            pl.BlockSpec((3, g, 128), lambda i: (0, 0, 0)),
        ],
        out_shape=[
            jax.ShapeDtypeStruct((np_, 128), F32),
            jax.ShapeDtypeStruct((3, g, 128), F32),
        ],
        compiler_params=pltpu.CompilerParams(
            dimension_semantics=("arbitrary",)),
    )
    return f(parts, hs, dinv, bt, batch3)


def _norm_scale_off(stats, gw, gb, gms):
    cnt = jnp.maximum(stats[2], 1.0)
    mean = stats[0] / cnt
    e2 = stats[1] / cnt
    ms = gms[0, :][None, :]
    var = e2 - mean * mean * ms * (2.0 - ms)
    rs = lax.rsqrt(var + EPS)
    w = gw[0, :][None, :]
    scale = w * rs
    off = gb[0, :][None, :] - w * ms * mean * rs
    return scale, off


def _row_gather(batch_ref, g, r, scale, off):
    b = batch_ref[0, 0, :]
    oht = (b[:, None] == lax.broadcasted_iota(jnp.int32, (r, g), 1)) \
        .astype(F32)
    return _dot(oht, scale), _dot(oht, off)


def _norm0_body(g, r, s_ref, stats_ref, batch_ref, dinv_ref,
                gw_ref, gb_ref, gms_ref, w1_ref, res_ref, hs1_ref):
    scale, off = _norm_scale_off(stats_ref[...], gw_ref, gb_ref, gms_ref)
    sc_r, off_r = _row_gather(batch_ref, g, r, scale, off)
    gv = _gelu(s_ref[...] * sc_r + off_r)
    res_ref[...] = gv
    hs1_ref[...] = dinv_ref[0, 0, :][:, None] * _dot(gv, w1_ref[...])


def _tc_norm0(g, np_, nblk, r, s, stats, batch3, dinv, gw, gb, gms, w1big):
    f = pl.pallas_call(
        functools.partial(_norm0_body, g, r),
        grid=(nblk,),
        in_specs=[
            pl.BlockSpec((r, 128), lambda i: (i, 0)),
            pl.BlockSpec((3, g, 128), lambda i: (0, 0, 0)),
            pl.BlockSpec((1, 1, r), lambda i: (i, 0, 0)),
            pl.BlockSpec((1, 1, r), lambda i: (i, 0, 0)),
            pl.BlockSpec((1, 128), lambda i: (0, 0)),
            pl.BlockSpec((1, 128), lambda i: (0, 0)),
            pl.BlockSpec((1, 128), lambda i: (0, 0)),
            pl.BlockSpec((128, 128), lambda i: (0, 0)),
        ],
        out_specs=[
            pl.BlockSpec((r, 128), lambda i: (i, 0)),
            pl.BlockSpec((r, 128), lambda i: (i, 0)),
        ],
        out_shape=[
            jax.ShapeDtypeStruct((np_, 128), F32),
            jax.ShapeDtypeStruct((np_, 128), F32),
        ],
        compiler_params=pltpu.CompilerParams(
            dimension_semantics=("arbitrary",)),
    )
    return f(s, stats, batch3, dinv, gw, gb, gms, w1big)


def _final_body(g, r, s_ref, stats_ref, batch_ref, res_ref,
                gw_ref, gb_ref, gms_ref, hw0_ref, hb0_ref, hw1_ref, hb1_ref,
                z_ref):
    scale, off = _norm_scale_off(stats_ref[...], gw_ref, gb_ref, gms_ref)
    sc_r, off_r = _row_gather(batch_ref, g, r, scale, off)
    h1 = _gelu(s_ref[...] * sc_r + off_r) + res_ref[...]
    t1 = _gelu(_dot(h1, hw0_ref[...]) + hb0_ref[0, :][None, :])
    z_ref[...] = _dot(t1, hw1_ref[...]) + hb1_ref[0, :][None, :]


def _tc_final(g, np_, nblk, r, s, stats, batch3, res0,
              gw, gb, gms, hw0big, hb0t, hw1big, hb1t):
    f = pl.pallas_call(
        functools.partial(_final_body, g, r),
        grid=(nblk,),
        in_specs=[
            pl.BlockSpec((r, 128), lambda i: (i, 0)),
            pl.BlockSpec((3, g, 128), lambda i: (0, 0, 0)),
            pl.BlockSpec((1, 1, r), lambda i: (i, 0, 0)),
            pl.BlockSpec((r, 128), lambda i: (i, 0)),
            pl.BlockSpec((1, 128), lambda i: (0, 0)),
            pl.BlockSpec((1, 128), lambda i: (0, 0)),
            pl.BlockSpec((1, 128), lambda i: (0, 0)),
            pl.BlockSpec((128, 128), lambda i: (0, 0)),
            pl.BlockSpec((1, 128), lambda i: (0, 0)),
            pl.BlockSpec((128, 128), lambda i: (0, 0)),
            pl.BlockSpec((1, 128), lambda i: (0, 0)),
        ],
        out_specs=[pl.BlockSpec((r, 128), lambda i: (i, 0))],
        out_shape=[jax.ShapeDtypeStruct((np_, 128), F32)],
        compiler_params=pltpu.CompilerParams(
            dimension_semantics=("arbitrary",)),
    )
    return f(s, stats, batch3, res0, gw, gb, gms, hw0big, hb0t, hw1big, hb1t)[0]


def kernel(x, batch, edge_index, W0, b0, gn0_w, gn0_b, gn0_ms,
           W1, b1, gn1_w, gn1_b, gn1_ms, hW0, hb0, hW1, hb1):
    n, t, d = x.shape
    h = W0.shape[1]
    e = edge_index.shape[1]
    g = 16
    out_f = hW1.shape[1]

    r = 1280
    np_ = ((n + 1 + r - 1) // r) * r
    nblk = np_ // r
    ncht = (e + NS * CHUNK - 1) // (NS * CHUNK)
    cpt0 = max(GRP, (ncht * 9) // 16 // GRP * GRP)
    cpt1 = max(GRP, (ncht - cpt0 + GRP - 1) // GRP * GRP)
    e_pad = NS * (cpt0 + cpt1) * CHUNK

    x2 = x.reshape(n, t * d)
    xp = jnp.zeros((np_, t * d), F32).at[:n].set(x2)
    batchp = jnp.full((np_,), g, jnp.int32).at[:n].set(batch.astype(jnp.int32))
    batch3 = batchp.reshape(nblk, 1, r)
    ei = edge_index.astype(jnp.int32)
    pad_idx = n + jnp.arange(e_pad - e, dtype=jnp.int32) % (np_ - n)
    srcp = jnp.concatenate([ei[0], pad_idx])
    dstp = jnp.concatenate([ei[1], pad_idx])

    srci = srcp.reshape(NS * (cpt0 + cpt1), CHUNK)
    dsti = dstp.reshape(NS * (cpt0 + cpt1), CHUNK)

    eye_t = jnp.eye(t, dtype=F32)
    w0big = jnp.kron(eye_t, W0)
    w1big = jnp.kron(eye_t, W1)
    hw0big = jnp.kron(eye_t, hW0)
    hw1big = jnp.zeros((t * h, 128), F32).at[:, :t * out_f].set(
        jnp.kron(eye_t, hW1))
    b0t = jnp.tile(b0, t).reshape(1, t * h)
    b1t = jnp.tile(b1, t).reshape(1, t * h)
    gw0 = jnp.tile(gn0_w, t).reshape(1, t * h)
    gb0 = jnp.tile(gn0_b, t).reshape(1, t * h)
    gm0 = jnp.tile(gn0_ms, t).reshape(1, t * h)
    gw1 = jnp.tile(gn1_w, t).reshape(1, t * h)
    gb1 = jnp.tile(gn1_b, t).reshape(1, t * h)
    gm1 = jnp.tile(gn1_ms, t).reshape(1, t * h)
    hb0t = jnp.tile(hb0, t).reshape(1, t * h)
    hb1t = jnp.zeros((1, 128), F32).at[0, :t * out_f].set(jnp.tile(hb1, t))

    rpw = np_ // NS
    zer128 = jnp.zeros((rpw, 128), F32)
    zer1 = jnp.zeros((640,), F32)
    ones1 = jnp.ones((CHUNK,), F32)

    degp = _sc_degree(cpt0, cpt1, dsti, ones1, zer1)
    deg0 = degp[0, 0, :np_].reshape(nblk, 1, r)
    deg1 = degp[1, 0, :np_].reshape(nblk, 1, r)

    hs0, dinv = _tc_scale(np_, nblk, r, xp, w0big, deg0, deg1)
    parts0 = _sc_aggregate(np_, cpt0, cpt1, hs0, srci, dsti, zer128)
    s0, stats0 = _tc_stats(g, np_, nblk, r, parts0, hs0, dinv, b0t, batch3)
    res0, hs1 = _tc_norm0(g, np_, nblk, r, s0, stats0, batch3, dinv,
                          gw0, gb0, gm0, w1big)
    parts1 = _sc_aggregate(np_, cpt0, cpt1, hs1, srci, dsti, zer128)
    s1, stats1 = _tc_stats(g, np_, nblk, r, parts1, hs1, dinv, b1t, batch3)
    z = _tc_final(g, np_, nblk, r, s1, stats1, batch3, res0,
                  gw1, gb1, gm1, hw0big, hb0t, hw1big, hb1t)

    return z[:n, :t * out_f].reshape(n, t, out_f)
